# Initial kernel scaffold; baseline (speedup 1.0000x reference)
import jax
import jax.numpy as jnp
from jax.experimental import pallas as pl


def kernel(x, pos, edge_index, edge_attr, batch, W_in, b_in, mu, We1, be1, W1, b1, We2, be2, W2, b2, We3, be3, W3, b3):
    raise NotImplementedError("write your pallas kernel here")


# speedup vs baseline: 2.0463x; 2.0463x over previous
import functools

import jax
import jax.numpy as jnp
from jax import lax
from jax.experimental import pallas as pl
from jax.experimental.pallas import tpu as pltpu
from jax.experimental.pallas import tpu_sc as plsc

NC = 2
NS = 16
LANES = 16
NW = NC * NS

GAMMA = 10.0
NG = 32


def _sc_d2(posx, posy, posz, rowi, coli, n_pad, cpt, ch):
    mesh = plsc.VectorSubcoreMesh(core_axis_name="c", subcore_axis_name="s")

    @functools.partial(
        pl.kernel,
        out_type=jax.ShapeDtypeStruct((NW, cpt, ch), jnp.float32),
        mesh=mesh,
        scratch_types=[
            pltpu.VMEM((n_pad,), jnp.float32),
            pltpu.VMEM((n_pad,), jnp.float32),
            pltpu.VMEM((n_pad,), jnp.float32),
            pltpu.VMEM((cpt, ch), jnp.int32),
            pltpu.VMEM((cpt, ch), jnp.int32),
            pltpu.VMEM((cpt, ch), jnp.float32),
        ],
        compiler_params=pltpu.CompilerParams(needs_layout_passes=False),
    )
    def run(px_hbm, py_hbm, pz_hbm, rowi_hbm, coli_hbm, d2_hbm,
            px, py, pz, row_v, col_v, d2_v):
        c = lax.axis_index("c")
        s = lax.axis_index("s")
        wid = c * NS + s
        pltpu.sync_copy(px_hbm, px)
        pltpu.sync_copy(py_hbm, py)
        pltpu.sync_copy(pz_hbm, pz)
        pltpu.sync_copy(rowi_hbm.at[wid], row_v)
        pltpu.sync_copy(coli_hbm.at[wid], col_v)

        def chunk(j, carry):
            def sub(k, carry2):
                sl = pl.ds(k * LANES, LANES)
                ri = row_v[j, sl]
                ci = col_v[j, sl]
                dx = plsc.load_gather(px, [ri]) - plsc.load_gather(px, [ci])
                dy = plsc.load_gather(py, [ri]) - plsc.load_gather(py, [ci])
                dz = plsc.load_gather(pz, [ri]) - plsc.load_gather(pz, [ci])
                d2_v[j, sl] = dx * dx + dy * dy + dz * dz
                return carry2

            return lax.fori_loop(0, ch // LANES, sub, carry)

        lax.fori_loop(0, cpt, chunk, 0)
        pltpu.sync_copy(d2_v, d2_hbm.at[wid])

    return run(posx, posy, posz, rowi, coli)


def _sc_conv(h_lo, h_hi, ef_lo, ef_hi, rowi, coli, zeros_h,
             n_pad, cpt, ch, dh):
    mesh = plsc.VectorSubcoreMesh(core_axis_name="c", subcore_axis_name="s")
    rps = n_pad // NS
    part_sd = jax.ShapeDtypeStruct((NC, n_pad, dh), jnp.float32)

    @functools.partial(
        pl.kernel,
        out_type=(part_sd, part_sd),
        mesh=mesh,
        scratch_types=[
            pltpu.VMEM((cpt, ch), jnp.int32),
            pltpu.VMEM((cpt, ch), jnp.int32),
            pltpu.VMEM((ch, dh), jnp.float32),
            pltpu.VMEM((ch, dh), jnp.float32),
            pltpu.VMEM_SHARED((n_pad, dh), jnp.float32),
        ],
        compiler_params=pltpu.CompilerParams(needs_layout_passes=False,
                                             use_tc_tiling_on_sc=False),
    )
    def run(hlo_hbm, hhi_hbm, eflo_hbm, efhi_hbm, rowi_hbm, coli_hbm, z_hbm,
            outlo_hbm, outhi_hbm, row_v, col_v, rows_v, ef_v, aggr):
        c = lax.axis_index("c")
        s = lax.axis_index("s")
        wid = c * NS + s
        mine = pl.ds(s * rps, rps)
        pltpu.sync_copy(rowi_hbm.at[wid], row_v)
        pltpu.sync_copy(coli_hbm.at[wid], col_v)

        for h_hbm, ef_hbm, out_hbm in ((hlo_hbm, eflo_hbm, outlo_hbm),
                                       (hhi_hbm, efhi_hbm, outhi_hbm)):
            pltpu.sync_copy(z_hbm.at[mine], aggr.at[mine])
            plsc.subcore_barrier()

            def chunk(j, carry):
                pltpu.sync_copy(h_hbm.at[row_v.at[j]], rows_v)
                pltpu.sync_copy(ef_hbm.at[pl.ds((wid * cpt + j) * ch, ch)], ef_v)

                def rloop(r, c2):
                    for q in range(dh // LANES):
                        sl = pl.ds(q * LANES, LANES)
                        rows_v[r, sl] = rows_v[r, sl] * ef_v[r, sl]
                    return c2

                lax.fori_loop(0, ch, rloop, 0)
                pltpu.sync_copy(rows_v, aggr.at[col_v.at[j]], add=True)
                return carry

            lax.fori_loop(0, cpt, chunk, 0)
            plsc.subcore_barrier()
            pltpu.sync_copy(aggr.at[mine], out_hbm.at[c, mine])
            plsc.subcore_barrier()

    return run(h_lo, h_hi, ef_lo, ef_hi, rowi, coli, zeros_h)


def _tc_ef(d2col, mu2, We1, We2, We3, be1, be2, be3, e_pad, nrbf, d, dh):
    eb = 1024
    grid = (e_pad // eb,)

    def body(d2_ref, mu_ref, w1, w2, w3, b1, b2, b3,
             o1l, o1h, o2l, o2h, o3l, o3h):
        dist = jnp.sqrt(d2_ref[...] + 1e-12)
        delta = dist - mu_ref[...]
        rbf = jnp.exp(-GAMMA * (delta * delta))
        for w, b, ol, oh in ((w1, b1, o1l, o1h), (w2, b2, o2l, o2h),
                             (w3, b3, o3l, o3h)):
            y = jnp.dot(rbf, w[...], preferred_element_type=jnp.float32) + b[...]
            ol[...] = y[:, :dh]
            oh[...] = y[:, dh:]

    full = lambda shape: pl.BlockSpec(shape, lambda i: (0,) * len(shape))
    out_sd = jax.ShapeDtypeStruct((e_pad, dh), jnp.float32)
    return pl.pallas_call(
        body,
        grid=grid,
        in_specs=[
            pl.BlockSpec((eb, 1), lambda i: (i, 0)),
            full((1, nrbf)),
            full((nrbf, d)), full((nrbf, d)), full((nrbf, d)),
            full((1, d)), full((1, d)), full((1, d)),
        ],
        out_specs=[pl.BlockSpec((eb, dh), lambda i: (i, 0))] * 6,
        out_shape=[out_sd] * 6,
    )(d2col, mu2, We1, We2, We3, be1, be2, be3)


def _tc_dense_in(a, W, b, n_pad, d, dh):
    rb = 512
    grid = (n_pad // rb,)

    def body(a_ref, w_ref, b_ref, ol_ref, oh_ref):
        y = (jnp.dot(a_ref[...], w_ref[...], preferred_element_type=jnp.float32)
             + b_ref[...])
        ol_ref[...] = y[:, :dh]
        oh_ref[...] = y[:, dh:]

    out_sd = jax.ShapeDtypeStruct((n_pad, dh), jnp.float32)
    return pl.pallas_call(
        body,
        grid=grid,
        in_specs=[
            pl.BlockSpec((rb, d), lambda i: (i, 0)),
            pl.BlockSpec((d, d), lambda i: (0, 0)),
            pl.BlockSpec((1, d), lambda i: (0, 0)),
        ],
        out_specs=[pl.BlockSpec((rb, dh), lambda i: (i, 0))] * 2,
        out_shape=[out_sd] * 2,
    )(a, W, b)


def _tc_dense_layer(p_lo, p_hi, W, b, n_pad, d, dh):
    rb = 512
    grid = (n_pad // rb,)

    def body(pl_ref, ph_ref, w_ref, b_ref, ol_ref, oh_ref):
        acc = jnp.concatenate(
            [jnp.sum(pl_ref[...], axis=0), jnp.sum(ph_ref[...], axis=0)],
            axis=1)
        y = jnp.dot(acc, w_ref[...], preferred_element_type=jnp.float32) + b_ref[...]
        y = jnp.maximum(y, 0.0)
        ol_ref[...] = y[:, :dh]
        oh_ref[...] = y[:, dh:]

    out_sd = jax.ShapeDtypeStruct((n_pad, dh), jnp.float32)
    return pl.pallas_call(
        body,
        grid=grid,
        in_specs=[
            pl.BlockSpec((NC, rb, dh), lambda i: (0, i, 0)),
            pl.BlockSpec((NC, rb, dh), lambda i: (0, i, 0)),
            pl.BlockSpec((d, d), lambda i: (0, 0)),
            pl.BlockSpec((1, d), lambda i: (0, 0)),
        ],
        out_specs=[pl.BlockSpec((rb, dh), lambda i: (i, 0))] * 2,
        out_shape=[out_sd] * 2,
    )(p_lo, p_hi, W, b)


def _tc_segmean(h_lo, h_hi, batch2, n_pad, d, dh):
    def body(b_ref, hl_ref, hh_ref, o_ref):
        seg = lax.broadcasted_iota(jnp.int32, (NG, n_pad), 0)
        m = (b_ref[...] == seg).astype(jnp.float32)
        sl = jnp.dot(m, hl_ref[...], preferred_element_type=jnp.float32)
        sh = jnp.dot(m, hh_ref[...], preferred_element_type=jnp.float32)
        cnt = jnp.sum(m, axis=1, keepdims=True)
        o_ref[...] = jnp.concatenate([sl, sh], axis=1) / jnp.maximum(cnt, 1.0)

    return pl.pallas_call(
        body,
        in_specs=[
            pl.BlockSpec((1, n_pad), lambda: (0, 0)),
            pl.BlockSpec((n_pad, dh), lambda: (0, 0)),
            pl.BlockSpec((n_pad, dh), lambda: (0, 0)),
        ],
        out_specs=pl.BlockSpec((NG, d), lambda: (0, 0)),
        out_shape=jax.ShapeDtypeStruct((NG, d), jnp.float32),
    )(batch2, h_lo, h_hi)


def kernel(x, pos, edge_index, edge_attr, batch, W_in, b_in, mu,
           We1, be1, W1, b1, We2, be2, W2, b2, We3, be3, W3, b3):
    n, d = x.shape
    e = edge_index.shape[1]
    nrbf = mu.shape[0]
    dh = d // 2

    ch = 128
    cpt = -(-e // (NW * ch))
    e_pad = NW * cpt * ch
    n_pad = -(-(n + 1) // 2048) * 2048

    row = edge_index[0]
    col = edge_index[1]
    pad_e = e_pad - e
    rowi = jnp.concatenate([row, jnp.zeros((pad_e,), jnp.int32)]).reshape(NW, cpt, ch)
    coli = jnp.concatenate([col, jnp.full((pad_e,), n, jnp.int32)]).reshape(NW, cpt, ch)
    posx = jnp.pad(pos[:, 0], (0, n_pad - n))
    posy = jnp.pad(pos[:, 1], (0, n_pad - n))
    posz = jnp.pad(pos[:, 2], (0, n_pad - n))
    x_pad = jnp.pad(x, ((0, n_pad - n), (0, 0)))
    zeros_h = jnp.zeros((n_pad, dh), jnp.float32)
    batch2 = jnp.pad(batch, (0, n_pad - n), constant_values=NG).reshape(1, n_pad)
    mu2 = mu.reshape(1, nrbf)

    d2 = _sc_d2(posx, posy, posz, rowi, coli, n_pad, cpt, ch)
    d2col = d2.reshape(e_pad, 1)
    efs = _tc_ef(d2col, mu2, We1, We2, We3,
                 be1.reshape(1, d), be2.reshape(1, d), be3.reshape(1, d),
                 e_pad, nrbf, d, dh)

    h_lo, h_hi = _tc_dense_in(x_pad, W_in, b_in.reshape(1, d), n_pad, d, dh)
    layers = ((efs[0], efs[1], W1, b1), (efs[2], efs[3], W2, b2),
              (efs[4], efs[5], W3, b3))
    for ef_lo, ef_hi, W, b in layers:
        p_lo, p_hi = _sc_conv(h_lo, h_hi, ef_lo, ef_hi, rowi, coli, zeros_h,
                              n_pad, cpt, ch, dh)
        h_lo, h_hi = _tc_dense_layer(p_lo, p_hi, W, b.reshape(1, d),
                                     n_pad, d, dh)

    return _tc_segmean(h_lo, h_hi, batch2, n_pad, d, dh)


# speedup vs baseline: 2.0713x; 1.0122x over previous
import functools

import jax
import jax.numpy as jnp
from jax import lax
from jax.experimental import pallas as pl
from jax.experimental.pallas import tpu as pltpu
from jax.experimental.pallas import tpu_sc as plsc

NC = 2
NS = 16
LANES = 16
NW = NC * NS

GAMMA = 10.0
NG = 32


def _sc_d2(posx, posy, posz, rowi, coli, n_pad, cpt, ch):
    mesh = plsc.VectorSubcoreMesh(core_axis_name="c", subcore_axis_name="s")

    @functools.partial(
        pl.kernel,
        out_type=jax.ShapeDtypeStruct((NW, cpt, ch), jnp.float32),
        mesh=mesh,
        scratch_types=[
            pltpu.VMEM((n_pad,), jnp.float32),
            pltpu.VMEM((n_pad,), jnp.float32),
            pltpu.VMEM((n_pad,), jnp.float32),
            pltpu.VMEM((cpt, ch), jnp.int32),
            pltpu.VMEM((cpt, ch), jnp.int32),
            pltpu.VMEM((cpt, ch), jnp.float32),
        ],
        compiler_params=pltpu.CompilerParams(needs_layout_passes=False),
    )
    def run(px_hbm, py_hbm, pz_hbm, rowi_hbm, coli_hbm, d2_hbm,
            px, py, pz, row_v, col_v, d2_v):
        c = lax.axis_index("c")
        s = lax.axis_index("s")
        wid = c * NS + s
        pltpu.sync_copy(px_hbm, px)
        pltpu.sync_copy(py_hbm, py)
        pltpu.sync_copy(pz_hbm, pz)
        pltpu.sync_copy(rowi_hbm.at[wid], row_v)
        pltpu.sync_copy(coli_hbm.at[wid], col_v)

        def chunk(j, carry):
            def sub(k, carry2):
                sl = pl.ds(k * LANES, LANES)
                ri = row_v[j, sl]
                ci = col_v[j, sl]
                dx = plsc.load_gather(px, [ri]) - plsc.load_gather(px, [ci])
                dy = plsc.load_gather(py, [ri]) - plsc.load_gather(py, [ci])
                dz = plsc.load_gather(pz, [ri]) - plsc.load_gather(pz, [ci])
                d2_v[j, sl] = dx * dx + dy * dy + dz * dz
                return carry2

            return lax.fori_loop(0, ch // LANES, sub, carry)

        lax.fori_loop(0, cpt, chunk, 0)
        pltpu.sync_copy(d2_v, d2_hbm.at[wid])

    return run(posx, posy, posz, rowi, coli)


def _sc_conv(h_lo, h_hi, ef_lo, ef_hi, rowi, coli, zeros_h,
             n_pad, cpt, ch, dh):
    mesh = plsc.VectorSubcoreMesh(core_axis_name="c", subcore_axis_name="s")
    rps = n_pad // NS
    part_sd = jax.ShapeDtypeStruct((NC, n_pad, dh), jnp.float32)
    buf = pltpu.VMEM((ch, dh), jnp.float32)

    @functools.partial(
        pl.kernel,
        out_type=(part_sd, part_sd),
        mesh=mesh,
        scratch_types=[
            pltpu.VMEM((cpt, ch), jnp.int32),
            pltpu.VMEM((cpt, ch), jnp.int32),
            buf, buf, buf, buf, buf, buf,
            pltpu.VMEM_SHARED((n_pad, dh), jnp.float32),
            pltpu.SemaphoreType.DMA, pltpu.SemaphoreType.DMA,
            pltpu.SemaphoreType.DMA, pltpu.SemaphoreType.DMA,
            pltpu.SemaphoreType.DMA, pltpu.SemaphoreType.DMA,
        ],
        compiler_params=pltpu.CompilerParams(needs_layout_passes=False,
                                             use_tc_tiling_on_sc=False),
    )
    def run(hlo_hbm, hhi_hbm, eflo_hbm, efhi_hbm, rowi_hbm, coli_hbm, z_hbm,
            outlo_hbm, outhi_hbm, row_v, col_v,
            rows_a, rows_b, efa, efb, msga, msgb, aggr,
            gs_a, gs_b, es_a, es_b, ss_a, ss_b):
        c = lax.axis_index("c")
        s = lax.axis_index("s")
        wid = c * NS + s
        mine = pl.ds(s * rps, rps)
        pltpu.sync_copy(rowi_hbm.at[wid], row_v)
        pltpu.sync_copy(coli_hbm.at[wid], col_v)
        slots = ((rows_a, efa, msga, gs_a, es_a, ss_a),
                 (rows_b, efb, msgb, gs_b, es_b, ss_b))

        for h_hbm, ef_hbm, out_hbm in ((hlo_hbm, eflo_hbm, outlo_hbm),
                                       (hhi_hbm, efhi_hbm, outhi_hbm)):
            pltpu.sync_copy(z_hbm.at[mine], aggr.at[mine])
            plsc.subcore_barrier()

            def start_fetch(j, rows_v, ef_v, gsem, esem):
                pltpu.async_copy(h_hbm.at[row_v.at[j]], rows_v, gsem)
                pltpu.async_copy(ef_hbm.at[pl.ds((wid * cpt + j) * ch, ch)],
                                 ef_v, esem)

            for b in range(2):
                rows_v, ef_v, _, gsem, esem, _ = slots[b]
                start_fetch(b, rows_v, ef_v, gsem, esem)

            def pair(j2, carry):
                for b in range(2):
                    rows_v, ef_v, msg_v, gsem, esem, ssem = slots[b]
                    j = 2 * j2 + b
                    pltpu.make_async_copy(h_hbm.at[row_v.at[j]], rows_v,
                                          gsem).wait()
                    pltpu.make_async_copy(
                        ef_hbm.at[pl.ds((wid * cpt + j) * ch, ch)], ef_v,
                        esem).wait()

                    @pl.when(j >= 2)
                    def _():
                        pltpu.make_async_copy(msg_v, aggr.at[col_v.at[j]],
                                              ssem).wait()

                    def rloop(r4, c2):
                        for rr in range(4):
                            r = r4 * 4 + rr
                            for q in range(dh // LANES):
                                sl = pl.ds(q * LANES, LANES)
                                msg_v[r, sl] = rows_v[r, sl] * ef_v[r, sl]
                        return c2

                    lax.fori_loop(0, ch // 4, rloop, 0)

                    @pl.when(j + 2 < cpt)
                    def _():
                        start_fetch(j + 2, rows_v, ef_v, gsem, esem)

                    pltpu.async_copy(msg_v, aggr.at[col_v.at[j]], ssem,
                                     add=True)
                return carry

            lax.fori_loop(0, cpt // 2, pair, 0)
            for b in range(2):
                _, _, msg_v, _, _, ssem = slots[b]
                j = cpt - 2 + b
                pltpu.make_async_copy(msg_v, aggr.at[col_v.at[j]], ssem).wait()
            plsc.subcore_barrier()
            pltpu.sync_copy(aggr.at[mine], out_hbm.at[c, mine])
            plsc.subcore_barrier()

    return run(h_lo, h_hi, ef_lo, ef_hi, rowi, coli, zeros_h)


def _tc_ef(d2col, mu2, We, be, e_pad, nrbf, d, dh):
    eb = 1024
    grid = (e_pad // eb,)

    def body(d2_ref, mu_ref, w, b, ol, oh):
        dist = jnp.sqrt(d2_ref[...] + 1e-12)
        delta = dist - mu_ref[...]
        rbf = jnp.exp(-GAMMA * (delta * delta))
        y = jnp.dot(rbf, w[...], preferred_element_type=jnp.float32) + b[...]
        ol[...] = y[:, :dh]
        oh[...] = y[:, dh:]

    full = lambda shape: pl.BlockSpec(shape, lambda i: (0,) * len(shape))
    out_sd = jax.ShapeDtypeStruct((e_pad, dh), jnp.float32)
    return pl.pallas_call(
        body,
        grid=grid,
        in_specs=[
            pl.BlockSpec((eb, 1), lambda i: (i, 0)),
            full((1, nrbf)),
            full((nrbf, d)),
            full((1, d)),
        ],
        out_specs=[pl.BlockSpec((eb, dh), lambda i: (i, 0))] * 2,
        out_shape=[out_sd] * 2,
    )(d2col, mu2, We, be)


def _tc_dense_in(a, W, b, n_pad, d, dh):
    rb = 512
    grid = (n_pad // rb,)

    def body(a_ref, w_ref, b_ref, ol_ref, oh_ref):
        y = (jnp.dot(a_ref[...], w_ref[...], preferred_element_type=jnp.float32)
             + b_ref[...])
        ol_ref[...] = y[:, :dh]
        oh_ref[...] = y[:, dh:]

    out_sd = jax.ShapeDtypeStruct((n_pad, dh), jnp.float32)
    return pl.pallas_call(
        body,
        grid=grid,
        in_specs=[
            pl.BlockSpec((rb, d), lambda i: (i, 0)),
            pl.BlockSpec((d, d), lambda i: (0, 0)),
            pl.BlockSpec((1, d), lambda i: (0, 0)),
        ],
        out_specs=[pl.BlockSpec((rb, dh), lambda i: (i, 0))] * 2,
        out_shape=[out_sd] * 2,
    )(a, W, b)


def _tc_dense_layer(p_lo, p_hi, W, b, n_pad, d, dh):
    rb = 512
    grid = (n_pad // rb,)

    def body(pl_ref, ph_ref, w_ref, b_ref, ol_ref, oh_ref):
        acc = jnp.concatenate(
            [jnp.sum(pl_ref[...], axis=0), jnp.sum(ph_ref[...], axis=0)],
            axis=1)
        y = jnp.dot(acc, w_ref[...], preferred_element_type=jnp.float32) + b_ref[...]
        y = jnp.maximum(y, 0.0)
        ol_ref[...] = y[:, :dh]
        oh_ref[...] = y[:, dh:]

    out_sd = jax.ShapeDtypeStruct((n_pad, dh), jnp.float32)
    return pl.pallas_call(
        body,
        grid=grid,
        in_specs=[
            pl.BlockSpec((NC, rb, dh), lambda i: (0, i, 0)),
            pl.BlockSpec((NC, rb, dh), lambda i: (0, i, 0)),
            pl.BlockSpec((d, d), lambda i: (0, 0)),
            pl.BlockSpec((1, d), lambda i: (0, 0)),
        ],
        out_specs=[pl.BlockSpec((rb, dh), lambda i: (i, 0))] * 2,
        out_shape=[out_sd] * 2,
    )(p_lo, p_hi, W, b)


def _tc_segmean(h_lo, h_hi, batch2, n_pad, d, dh):
    def body(b_ref, hl_ref, hh_ref, o_ref):
        seg = lax.broadcasted_iota(jnp.int32, (NG, n_pad), 0)
        m = (b_ref[...] == seg).astype(jnp.float32)
        sl = jnp.dot(m, hl_ref[...], preferred_element_type=jnp.float32)
        sh = jnp.dot(m, hh_ref[...], preferred_element_type=jnp.float32)
        cnt = jnp.sum(m, axis=1, keepdims=True)
        o_ref[...] = jnp.concatenate([sl, sh], axis=1) / jnp.maximum(cnt, 1.0)

    return pl.pallas_call(
        body,
        in_specs=[
            pl.BlockSpec((1, n_pad), lambda: (0, 0)),
            pl.BlockSpec((n_pad, dh), lambda: (0, 0)),
            pl.BlockSpec((n_pad, dh), lambda: (0, 0)),
        ],
        out_specs=pl.BlockSpec((NG, d), lambda: (0, 0)),
        out_shape=jax.ShapeDtypeStruct((NG, d), jnp.float32),
    )(batch2, h_lo, h_hi)


def kernel(x, pos, edge_index, edge_attr, batch, W_in, b_in, mu,
           We1, be1, W1, b1, We2, be2, W2, b2, We3, be3, W3, b3):
    n, d = x.shape
    e = edge_index.shape[1]
    nrbf = mu.shape[0]
    dh = d // 2

    ch = 128
    cpt = -(-e // (NW * ch))
    cpt += cpt % 2
    e_pad = NW * cpt * ch
    n_pad = -(-(n + 1) // 2048) * 2048

    row = edge_index[0]
    col = edge_index[1]
    pad_e = e_pad - e
    rowi = jnp.concatenate([row, jnp.zeros((pad_e,), jnp.int32)]).reshape(NW, cpt, ch)
    dummy = n + jnp.arange(pad_e, dtype=jnp.int32) % (n_pad - n)
    coli = jnp.concatenate([col, dummy]).reshape(NW, cpt, ch)
    posx = jnp.pad(pos[:, 0], (0, n_pad - n))
    posy = jnp.pad(pos[:, 1], (0, n_pad - n))
    posz = jnp.pad(pos[:, 2], (0, n_pad - n))
    x_pad = jnp.pad(x, ((0, n_pad - n), (0, 0)))
    zeros_h = jnp.zeros((n_pad, dh), jnp.float32)
    batch2 = jnp.pad(batch, (0, n_pad - n), constant_values=NG).reshape(1, n_pad)
    mu2 = mu.reshape(1, nrbf)

    d2 = _sc_d2(posx, posy, posz, rowi, coli, n_pad, cpt, ch)
    d2col = d2.reshape(e_pad, 1)

    h_lo, h_hi = _tc_dense_in(x_pad, W_in, b_in.reshape(1, d), n_pad, d, dh)
    for We, be, W, b in ((We1, be1, W1, b1), (We2, be2, W2, b2),
                         (We3, be3, W3, b3)):
        ef_lo, ef_hi = _tc_ef(d2col, mu2, We, be.reshape(1, d),
                              e_pad, nrbf, d, dh)
        p_lo, p_hi = _sc_conv(h_lo, h_hi, ef_lo, ef_hi, rowi, coli, zeros_h,
                              n_pad, cpt, ch, dh)
        h_lo, h_hi = _tc_dense_layer(p_lo, p_hi, W, b.reshape(1, d),
                                     n_pad, d, dh)

    return _tc_segmean(h_lo, h_hi, batch2, n_pad, d, dh)


# speedup vs baseline: 2.7758x; 1.3401x over previous
import functools

import jax
import jax.numpy as jnp
from jax import lax
from jax.experimental import pallas as pl
from jax.experimental.pallas import tpu as pltpu
from jax.experimental.pallas import tpu_sc as plsc

NC = 2
NS = 16
LANES = 16
NW = NC * NS

GAMMA = 10.0
NG = 32


def _sc_d2(posx, posy, posz, rowi, coli, n_pad, cpt, ch):
    mesh = plsc.VectorSubcoreMesh(core_axis_name="c", subcore_axis_name="s")

    @functools.partial(
        pl.kernel,
        out_type=jax.ShapeDtypeStruct((NW, cpt, ch), jnp.float32),
        mesh=mesh,
        scratch_types=[
            pltpu.VMEM((n_pad,), jnp.float32),
            pltpu.VMEM((n_pad,), jnp.float32),
            pltpu.VMEM((n_pad,), jnp.float32),
            pltpu.VMEM((cpt, ch), jnp.int32),
            pltpu.VMEM((cpt, ch), jnp.int32),
            pltpu.VMEM((cpt, ch), jnp.float32),
        ],
        compiler_params=pltpu.CompilerParams(needs_layout_passes=False),
    )
    def run(px_hbm, py_hbm, pz_hbm, rowi_hbm, coli_hbm, d2_hbm,
            px, py, pz, row_v, col_v, d2_v):
        c = lax.axis_index("c")
        s = lax.axis_index("s")
        wid = c * NS + s
        pltpu.sync_copy(px_hbm, px)
        pltpu.sync_copy(py_hbm, py)
        pltpu.sync_copy(pz_hbm, pz)
        pltpu.sync_copy(rowi_hbm.at[wid], row_v)
        pltpu.sync_copy(coli_hbm.at[wid], col_v)

        def chunk(j, carry):
            def sub(k, carry2):
                sl = pl.ds(k * LANES, LANES)
                ri = row_v[j, sl]
                ci = col_v[j, sl]
                dx = plsc.load_gather(px, [ri]) - plsc.load_gather(px, [ci])
                dy = plsc.load_gather(py, [ri]) - plsc.load_gather(py, [ci])
                dz = plsc.load_gather(pz, [ri]) - plsc.load_gather(pz, [ci])
                d2_v[j, sl] = dx * dx + dy * dy + dz * dz
                return carry2

            return lax.fori_loop(0, ch // LANES, sub, carry)

        lax.fori_loop(0, cpt, chunk, 0)
        pltpu.sync_copy(d2_v, d2_hbm.at[wid])

    return run(posx, posy, posz, rowi, coli)


def _sc_conv(h_lo, h_hi, ef_lo, ef_hi, rowi, coli, zeros_h,
             n_pad, cpt, ch, dh):
    mesh = plsc.VectorSubcoreMesh(core_axis_name="c", subcore_axis_name="s")
    rps = n_pad // NS
    part_sd = jax.ShapeDtypeStruct((NC, n_pad, dh), jnp.float32)
    buf = pltpu.VMEM((ch, dh), jnp.float32)

    @functools.partial(
        pl.kernel,
        out_type=(part_sd, part_sd),
        mesh=mesh,
        scratch_types=[
            pltpu.VMEM((cpt, ch), jnp.int32),
            pltpu.VMEM((cpt, ch), jnp.int32),
            buf, buf, buf, buf, buf, buf,
            pltpu.VMEM_SHARED((n_pad, dh), jnp.float32),
            pltpu.SemaphoreType.DMA, pltpu.SemaphoreType.DMA,
            pltpu.SemaphoreType.DMA, pltpu.SemaphoreType.DMA,
            pltpu.SemaphoreType.DMA, pltpu.SemaphoreType.DMA,
        ],
        compiler_params=pltpu.CompilerParams(needs_layout_passes=False,
                                             use_tc_tiling_on_sc=False),
    )
    def run(hlo_hbm, hhi_hbm, eflo_hbm, efhi_hbm, rowi_hbm, coli_hbm, z_hbm,
            outlo_hbm, outhi_hbm, row_v, col_v,
            rows_a, rows_b, efa, efb, msga, msgb, aggr,
            gs_a, gs_b, es_a, es_b, ss_a, ss_b):
        c = lax.axis_index("c")
        s = lax.axis_index("s")
        wid = c * NS + s
        mine = pl.ds(s * rps, rps)
        pltpu.sync_copy(rowi_hbm.at[wid], row_v)
        pltpu.sync_copy(coli_hbm.at[wid], col_v)
        slots = ((rows_a, efa, msga, gs_a, es_a, ss_a),
                 (rows_b, efb, msgb, gs_b, es_b, ss_b))

        for h_hbm, ef_hbm, out_hbm in ((hlo_hbm, eflo_hbm, outlo_hbm),
                                       (hhi_hbm, efhi_hbm, outhi_hbm)):
            pltpu.sync_copy(z_hbm.at[mine], aggr.at[mine])
            plsc.subcore_barrier()

            def start_fetch(j, rows_v, ef_v, gsem, esem):
                pltpu.async_copy(h_hbm.at[row_v.at[j]], rows_v, gsem)
                pltpu.async_copy(ef_hbm.at[pl.ds((wid * cpt + j) * ch, ch)],
                                 ef_v, esem)

            for b in range(2):
                rows_v, ef_v, _, gsem, esem, _ = slots[b]
                start_fetch(b, rows_v, ef_v, gsem, esem)

            def pair(j2, carry):
                for b in range(2):
                    rows_v, ef_v, msg_v, gsem, esem, ssem = slots[b]
                    j = 2 * j2 + b
                    pltpu.make_async_copy(h_hbm.at[row_v.at[j]], rows_v,
                                          gsem).wait()
                    pltpu.make_async_copy(
                        ef_hbm.at[pl.ds((wid * cpt + j) * ch, ch)], ef_v,
                        esem).wait()

                    @pl.when(j >= 2)
                    def _():
                        pltpu.make_async_copy(msg_v, aggr.at[col_v.at[j]],
                                              ssem).wait()

                    def rloop(r4, c2):
                        for rr in range(4):
                            r = r4 * 4 + rr
                            for q in range(dh // LANES):
                                sl = pl.ds(q * LANES, LANES)
                                msg_v[r, sl] = rows_v[r, sl] * ef_v[r, sl]
                        return c2

                    lax.fori_loop(0, ch // 4, rloop, 0)

                    @pl.when(j + 2 < cpt)
                    def _():
                        start_fetch(j + 2, rows_v, ef_v, gsem, esem)

                    pltpu.async_copy(msg_v, aggr.at[col_v.at[j]], ssem,
                                     add=True)
                return carry

            lax.fori_loop(0, cpt // 2, pair, 0)
            for b in range(2):
                _, _, msg_v, _, _, ssem = slots[b]
                j = cpt - 2 + b
                pltpu.make_async_copy(msg_v, aggr.at[col_v.at[j]], ssem).wait()
            plsc.subcore_barrier()
            pltpu.sync_copy(aggr.at[mine], out_hbm.at[c, mine])
            plsc.subcore_barrier()

    return run(h_lo, h_hi, ef_lo, ef_hi, rowi, coli, zeros_h)


def _tc_ef(d2col, mu2, We, be, e_pad, nrbf, d, dh):
    eb = 1024
    grid = (e_pad // eb,)

    def body(d2_ref, mu_ref, w, b, ol, oh):
        dist = jnp.sqrt(d2_ref[...] + 1e-12)
        delta = dist - mu_ref[...]
        rbf = jnp.exp(-GAMMA * (delta * delta))
        y = jnp.dot(rbf, w[...], preferred_element_type=jnp.float32) + b[...]
        ol[...] = y[:, :dh]
        oh[...] = y[:, dh:]

    full = lambda shape: pl.BlockSpec(shape, lambda i: (0,) * len(shape))
    out_sd = jax.ShapeDtypeStruct((e_pad, dh), jnp.float32)
    return pl.pallas_call(
        body,
        grid=grid,
        in_specs=[
            pl.BlockSpec((eb, 1), lambda i: (i, 0)),
            full((1, nrbf)),
            full((nrbf, d)),
            full((1, d)),
        ],
        out_specs=[pl.BlockSpec((eb, dh), lambda i: (i, 0))] * 2,
        out_shape=[out_sd] * 2,
    )(d2col, mu2, We, be)


def _tc_dense_in(a, W, b, n_pad, d, dh):
    rb = 512
    grid = (n_pad // rb,)

    def body(a_ref, w_ref, b_ref, ol_ref, oh_ref):
        y = (jnp.dot(a_ref[...], w_ref[...], preferred_element_type=jnp.float32)
             + b_ref[...])
        ol_ref[...] = y[:, :dh]
        oh_ref[...] = y[:, dh:]

    out_sd = jax.ShapeDtypeStruct((n_pad, dh), jnp.float32)
    return pl.pallas_call(
        body,
        grid=grid,
        in_specs=[
            pl.BlockSpec((rb, d), lambda i: (i, 0)),
            pl.BlockSpec((d, d), lambda i: (0, 0)),
            pl.BlockSpec((1, d), lambda i: (0, 0)),
        ],
        out_specs=[pl.BlockSpec((rb, dh), lambda i: (i, 0))] * 2,
        out_shape=[out_sd] * 2,
    )(a, W, b)


def _tc_dense_layer(p_lo, p_hi, W, b, n_pad, d, dh):
    rb = 512
    grid = (n_pad // rb,)

    def body(pl_ref, ph_ref, w_ref, b_ref, ol_ref, oh_ref):
        acc = jnp.concatenate(
            [jnp.sum(pl_ref[...], axis=0), jnp.sum(ph_ref[...], axis=0)],
            axis=1)
        y = jnp.dot(acc, w_ref[...], preferred_element_type=jnp.float32) + b_ref[...]
        y = jnp.maximum(y, 0.0)
        ol_ref[...] = y[:, :dh]
        oh_ref[...] = y[:, dh:]

    out_sd = jax.ShapeDtypeStruct((n_pad, dh), jnp.float32)
    return pl.pallas_call(
        body,
        grid=grid,
        in_specs=[
            pl.BlockSpec((NC, rb, dh), lambda i: (0, i, 0)),
            pl.BlockSpec((NC, rb, dh), lambda i: (0, i, 0)),
            pl.BlockSpec((d, d), lambda i: (0, 0)),
            pl.BlockSpec((1, d), lambda i: (0, 0)),
        ],
        out_specs=[pl.BlockSpec((rb, dh), lambda i: (i, 0))] * 2,
        out_shape=[out_sd] * 2,
    )(p_lo, p_hi, W, b)


def _tc_segmean(h_lo, h_hi, batch2, n_pad, d, dh):
    rb = 1024
    grid = (n_pad // rb,)

    def body(b_ref, hl_ref, hh_ref, o_ref, acc, cacc):
        i = pl.program_id(0)
        seg = lax.broadcasted_iota(jnp.int32, (NG, rb), 0)
        m = (b_ref[...] == seg).astype(jnp.float32)
        sl = jnp.dot(m, hl_ref[...], preferred_element_type=jnp.float32)
        sh = jnp.dot(m, hh_ref[...], preferred_element_type=jnp.float32)
        sums = jnp.concatenate([sl, sh], axis=1)
        cnt = jnp.sum(m, axis=1, keepdims=True)

        @pl.when(i == 0)
        def _():
            acc[...] = jnp.zeros_like(acc)
            cacc[...] = jnp.zeros_like(cacc)

        acc[...] = acc[...] + sums
        cacc[...] = cacc[...] + cnt

        @pl.when(i == grid[0] - 1)
        def _():
            o_ref[...] = acc[...] / jnp.maximum(cacc[...], 1.0)

    return pl.pallas_call(
        body,
        grid=grid,
        in_specs=[
            pl.BlockSpec((1, rb), lambda i: (0, i)),
            pl.BlockSpec((rb, dh), lambda i: (i, 0)),
            pl.BlockSpec((rb, dh), lambda i: (i, 0)),
        ],
        out_specs=pl.BlockSpec((NG, d), lambda i: (0, 0)),
        out_shape=jax.ShapeDtypeStruct((NG, d), jnp.float32),
        scratch_shapes=[
            pltpu.VMEM((NG, d), jnp.float32),
            pltpu.VMEM((NG, 1), jnp.float32),
        ],
    )(batch2, h_lo, h_hi)


def kernel(x, pos, edge_index, edge_attr, batch, W_in, b_in, mu,
           We1, be1, W1, b1, We2, be2, W2, b2, We3, be3, W3, b3):
    n, d = x.shape
    e = edge_index.shape[1]
    nrbf = mu.shape[0]
    dh = d // 2

    ch = 128
    cpt = -(-e // (NW * ch))
    cpt += cpt % 2
    e_pad = NW * cpt * ch
    n_pad = -(-(n + 1) // 2048) * 2048

    row = edge_index[0]
    col = edge_index[1]
    e_pt = -(-e // NW)
    ep = cpt * ch
    row_p = jnp.concatenate([row, jnp.zeros((e_pt * NW - e,), jnp.int32)])
    col_p = jnp.concatenate([col, jnp.full((e_pt * NW - e,), n, jnp.int32)])
    pad_rows = jnp.broadcast_to(
        jnp.arange(ep - e_pt, dtype=jnp.int32) % n, (NW, ep - e_pt))
    pad_cols = jnp.broadcast_to(
        n + jnp.arange(ep - e_pt, dtype=jnp.int32) % (n_pad - n),
        (NW, ep - e_pt))
    rowi = jnp.concatenate(
        [row_p.reshape(NW, e_pt), pad_rows], axis=1).reshape(NW, cpt, ch)
    coli = jnp.concatenate(
        [col_p.reshape(NW, e_pt), pad_cols], axis=1).reshape(NW, cpt, ch)
    posx = jnp.pad(pos[:, 0], (0, n_pad - n))
    posy = jnp.pad(pos[:, 1], (0, n_pad - n))
    posz = jnp.pad(pos[:, 2], (0, n_pad - n))
    x_pad = jnp.pad(x, ((0, n_pad - n), (0, 0)))
    zeros_h = jnp.zeros((n_pad, dh), jnp.float32)
    batch2 = jnp.pad(batch, (0, n_pad - n), constant_values=NG).reshape(1, n_pad)
    mu2 = mu.reshape(1, nrbf)

    d2 = _sc_d2(posx, posy, posz, rowi, coli, n_pad, cpt, ch)
    d2col = d2.reshape(e_pad, 1)

    h_lo, h_hi = _tc_dense_in(x_pad, W_in, b_in.reshape(1, d), n_pad, d, dh)
    for We, be, W, b in ((We1, be1, W1, b1), (We2, be2, W2, b2),
                         (We3, be3, W3, b3)):
        ef_lo, ef_hi = _tc_ef(d2col, mu2, We, be.reshape(1, d),
                              e_pad, nrbf, d, dh)
        p_lo, p_hi = _sc_conv(h_lo, h_hi, ef_lo, ef_hi, rowi, coli, zeros_h,
                              n_pad, cpt, ch, dh)
        h_lo, h_hi = _tc_dense_layer(p_lo, p_hi, W, b.reshape(1, d),
                                     n_pad, d, dh)

    return _tc_segmean(h_lo, h_hi, batch2, n_pad, d, dh)


# speedup vs baseline: 5.9030x; 2.1266x over previous
import functools

import jax
import jax.numpy as jnp
from jax import lax
from jax.experimental import pallas as pl
from jax.experimental.pallas import tpu as pltpu
from jax.experimental.pallas import tpu_sc as plsc

NC = 2
NS = 16
LANES = 16
NW = NC * NS

GAMMA = 10.0
NG = 32


def _sc_d2(posx, posy, posz, rowi, coli, n_pad, cpt, ch):
    mesh = plsc.VectorSubcoreMesh(core_axis_name="c", subcore_axis_name="s")

    @functools.partial(
        pl.kernel,
        out_type=jax.ShapeDtypeStruct((NW, cpt, ch), jnp.float32),
        mesh=mesh,
        scratch_types=[
            pltpu.VMEM((n_pad,), jnp.float32),
            pltpu.VMEM((n_pad,), jnp.float32),
            pltpu.VMEM((n_pad,), jnp.float32),
            pltpu.VMEM((cpt, ch), jnp.int32),
            pltpu.VMEM((cpt, ch), jnp.int32),
            pltpu.VMEM((cpt, ch), jnp.float32),
        ],
        compiler_params=pltpu.CompilerParams(needs_layout_passes=False),
    )
    def run(px_hbm, py_hbm, pz_hbm, rowi_hbm, coli_hbm, d2_hbm,
            px, py, pz, row_v, col_v, d2_v):
        c = lax.axis_index("c")
        s = lax.axis_index("s")
        wid = c * NS + s
        pltpu.sync_copy(px_hbm, px)
        pltpu.sync_copy(py_hbm, py)
        pltpu.sync_copy(pz_hbm, pz)
        pltpu.sync_copy(rowi_hbm.at[wid], row_v)
        pltpu.sync_copy(coli_hbm.at[wid], col_v)

        def chunk(j, carry):
            def sub(k, carry2):
                sl = pl.ds(k * LANES, LANES)
                ri = row_v[j, sl]
                ci = col_v[j, sl]
                dx = plsc.load_gather(px, [ri]) - plsc.load_gather(px, [ci])
                dy = plsc.load_gather(py, [ri]) - plsc.load_gather(py, [ci])
                dz = plsc.load_gather(pz, [ri]) - plsc.load_gather(pz, [ci])
                d2_v[j, sl] = dx * dx + dy * dy + dz * dz
                return carry2

            return lax.fori_loop(0, ch // LANES, sub, carry)

        lax.fori_loop(0, cpt, chunk, 0)
        pltpu.sync_copy(d2_v, d2_hbm.at[wid])

    return run(posx, posy, posz, rowi, coli)


def _sc_conv(h_lo, h_hi, ef, rowi, coli, zeros_h, n_pad, cpt, ch, d, dh):
    mesh = plsc.VectorSubcoreMesh(core_axis_name="c", subcore_axis_name="s")
    rps = n_pad // NS
    buf = pltpu.VMEM((ch, dh), jnp.float32)
    part_sd = jax.ShapeDtypeStruct((NC, n_pad, dh), jnp.float32)

    @functools.partial(
        pl.kernel,
        out_type=(part_sd, part_sd),
        mesh=mesh,
        scratch_types=[
            pltpu.VMEM((cpt, ch), jnp.int32),
            pltpu.VMEM((cpt, ch), jnp.int32),
            buf, buf, buf, buf, buf, buf,
            pltpu.VMEM_SHARED((n_pad, dh), jnp.float32),
            pltpu.SemaphoreType.DMA, pltpu.SemaphoreType.DMA,
            pltpu.SemaphoreType.DMA, pltpu.SemaphoreType.DMA,
            pltpu.SemaphoreType.DMA, pltpu.SemaphoreType.DMA,
        ],
        compiler_params=pltpu.CompilerParams(needs_layout_passes=False,
                                             use_tc_tiling_on_sc=False),
    )
    def run(hlo_hbm, hhi_hbm, ef_hbm, rowi_hbm, coli_hbm, z_hbm,
            outlo_hbm, outhi_hbm,
            row_v, col_v, rows_a, rows_b, efa, efb, msga, msgb, aggr,
            gs_a, gs_b, es_a, es_b, ss_a, ss_b):
        c = lax.axis_index("c")
        s = lax.axis_index("s")
        wid = c * NS + s
        mine = pl.ds(s * rps, rps)
        pltpu.sync_copy(rowi_hbm.at[wid], row_v)
        pltpu.sync_copy(coli_hbm.at[wid], col_v)
        slots = ((rows_a, efa, msga, gs_a, es_a, ss_a),
                 (rows_b, efb, msgb, gs_b, es_b, ss_b))

        for hx, (h_hbm, out_hbm) in enumerate(((hlo_hbm, outlo_hbm),
                                               (hhi_hbm, outhi_hbm))):
            off = pl.ds(hx * dh, dh)
            pltpu.sync_copy(z_hbm.at[mine], aggr.at[mine])
            plsc.subcore_barrier()

            def start_fetch(j, rows_v, ef_v, gsem, esem):
                pltpu.async_copy(h_hbm.at[row_v.at[j]], rows_v, gsem)
                pltpu.async_copy(
                    ef_hbm.at[pl.ds((wid * cpt + j) * ch, ch), off],
                    ef_v, esem)

            for b in range(2):
                rows_v, ef_v, _, gsem, esem, _ = slots[b]
                start_fetch(b, rows_v, ef_v, gsem, esem)

            def pair(j2, carry):
                for b in range(2):
                    rows_v, ef_v, msg_v, gsem, esem, ssem = slots[b]
                    j = 2 * j2 + b
                    pltpu.make_async_copy(h_hbm.at[row_v.at[j]], rows_v,
                                          gsem).wait()
                    pltpu.make_async_copy(
                        ef_hbm.at[pl.ds((wid * cpt + j) * ch, ch), off],
                        ef_v, esem).wait()

                    @pl.when(j >= 2)
                    def _():
                        pltpu.make_async_copy(msg_v, aggr.at[col_v.at[j]],
                                              ssem).wait()

                    def rloop(r4, c2):
                        for rr in range(4):
                            r = r4 * 4 + rr
                            for q in range(dh // LANES):
                                sl = pl.ds(q * LANES, LANES)
                                msg_v[r, sl] = rows_v[r, sl] * ef_v[r, sl]
                        return c2

                    lax.fori_loop(0, ch // 4, rloop, 0)

                    @pl.when(j + 2 < cpt)
                    def _():
                        start_fetch(j + 2, rows_v, ef_v, gsem, esem)

                    pltpu.async_copy(msg_v, aggr.at[col_v.at[j]], ssem,
                                     add=True)
                return carry

            lax.fori_loop(0, cpt // 2, pair, 0)
            for b in range(2):
                _, _, msg_v, _, _, ssem = slots[b]
                j = cpt - 2 + b
                pltpu.make_async_copy(msg_v, aggr.at[col_v.at[j]], ssem).wait()
            plsc.subcore_barrier()
            pltpu.sync_copy(aggr.at[mine], out_hbm.at[c, mine])
            plsc.subcore_barrier()

    return run(h_lo, h_hi, ef, rowi, coli, zeros_h)


def _tc_ef(d2m, mu_col, We, be, e_pad, nrbf, d):
    eb = 1024
    sb = eb // 128
    grid = (e_pad // eb,)

    def body(d2_ref, mu_ref, w, b, o):
        for s in range(sb):
            dist = jnp.sqrt(d2_ref[s:s + 1, :] + 1e-12)
            delta = dist - mu_ref[...]
            rbf_t = jnp.exp(-GAMMA * (delta * delta))
            y = lax.dot_general(rbf_t, w[...], (((0,), (0,)), ((), ())),
                                preferred_element_type=jnp.float32)
            o[s] = y + b[...]

    full = lambda shape: pl.BlockSpec(shape, lambda i: (0,) * len(shape))
    return pl.pallas_call(
        body,
        grid=grid,
        in_specs=[
            pl.BlockSpec((sb, 128), lambda i: (i, 0)),
            full((nrbf, 1)),
            full((nrbf, d)),
            full((1, d)),
        ],
        out_specs=pl.BlockSpec((sb, 128, d), lambda i: (i, 0, 0)),
        out_shape=jax.ShapeDtypeStruct((e_pad // 128, 128, d), jnp.float32),
    )(d2m, mu_col, We, be)


def _tc_dense_in(a, W, b, n_pad, d, dh):
    rb = 512
    grid = (n_pad // rb,)

    def body(a_ref, w_ref, b_ref, ol_ref, oh_ref):
        y = (jnp.dot(a_ref[...], w_ref[...], preferred_element_type=jnp.float32)
             + b_ref[...])
        ol_ref[...] = y[:, :dh]
        oh_ref[...] = y[:, dh:]

    out_sd = jax.ShapeDtypeStruct((n_pad, dh), jnp.float32)
    return pl.pallas_call(
        body,
        grid=grid,
        in_specs=[
            pl.BlockSpec((rb, d), lambda i: (i, 0)),
            pl.BlockSpec((d, d), lambda i: (0, 0)),
            pl.BlockSpec((1, d), lambda i: (0, 0)),
        ],
        out_specs=[pl.BlockSpec((rb, dh), lambda i: (i, 0))] * 2,
        out_shape=[out_sd] * 2,
    )(a, W, b)


def _tc_dense_layer(p_lo, p_hi, W, b, n_pad, d, dh):
    rb = 512
    grid = (n_pad // rb,)

    def body(pl_ref, ph_ref, w_ref, b_ref, ol_ref, oh_ref):
        acc = jnp.concatenate(
            [jnp.sum(pl_ref[...], axis=0), jnp.sum(ph_ref[...], axis=0)],
            axis=1)
        y = jnp.dot(acc, w_ref[...], preferred_element_type=jnp.float32) + b_ref[...]
        y = jnp.maximum(y, 0.0)
        ol_ref[...] = y[:, :dh]
        oh_ref[...] = y[:, dh:]

    out_sd = jax.ShapeDtypeStruct((n_pad, dh), jnp.float32)
    return pl.pallas_call(
        body,
        grid=grid,
        in_specs=[
            pl.BlockSpec((NC, rb, dh), lambda i: (0, i, 0)),
            pl.BlockSpec((NC, rb, dh), lambda i: (0, i, 0)),
            pl.BlockSpec((d, d), lambda i: (0, 0)),
            pl.BlockSpec((1, d), lambda i: (0, 0)),
        ],
        out_specs=[pl.BlockSpec((rb, dh), lambda i: (i, 0))] * 2,
        out_shape=[out_sd] * 2,
    )(p_lo, p_hi, W, b)


def _tc_segmean(h_lo, h_hi, batch2, n_pad, d, dh):
    rb = 1024
    grid = (n_pad // rb,)

    def body(b_ref, hl_ref, hh_ref, o_ref, acc, cacc):
        i = pl.program_id(0)
        seg = lax.broadcasted_iota(jnp.int32, (NG, rb), 0)
        m = (b_ref[...] == seg).astype(jnp.float32)
        sl = jnp.dot(m, hl_ref[...], preferred_element_type=jnp.float32)
        sh = jnp.dot(m, hh_ref[...], preferred_element_type=jnp.float32)
        sums = jnp.concatenate([sl, sh], axis=1)
        cnt = jnp.sum(m, axis=1, keepdims=True)

        @pl.when(i == 0)
        def _():
            acc[...] = jnp.zeros_like(acc)
            cacc[...] = jnp.zeros_like(cacc)

        acc[...] = acc[...] + sums
        cacc[...] = cacc[...] + cnt

        @pl.when(i == grid[0] - 1)
        def _():
            o_ref[...] = acc[...] / jnp.maximum(cacc[...], 1.0)

    return pl.pallas_call(
        body,
        grid=grid,
        in_specs=[
            pl.BlockSpec((1, rb), lambda i: (0, i)),
            pl.BlockSpec((rb, dh), lambda i: (i, 0)),
            pl.BlockSpec((rb, dh), lambda i: (i, 0)),
        ],
        out_specs=pl.BlockSpec((NG, d), lambda i: (0, 0)),
        out_shape=jax.ShapeDtypeStruct((NG, d), jnp.float32),
        scratch_shapes=[
            pltpu.VMEM((NG, d), jnp.float32),
            pltpu.VMEM((NG, 1), jnp.float32),
        ],
    )(batch2, h_lo, h_hi)


def kernel(x, pos, edge_index, edge_attr, batch, W_in, b_in, mu,
           We1, be1, W1, b1, We2, be2, W2, b2, We3, be3, W3, b3):
    n, d = x.shape
    e = edge_index.shape[1]
    nrbf = mu.shape[0]
    dh = d // 2

    ch = 128
    cpt = -(-e // (NW * ch))
    cpt += cpt % 2
    e_pad = NW * cpt * ch
    n_pad = -(-(n + 1) // 2048) * 2048

    row = edge_index[0]
    col = edge_index[1]
    e_pt = -(-e // NW)
    row_p = jnp.concatenate([row, jnp.zeros((e_pt * NW - e,), jnp.int32)])
    col_p = jnp.concatenate([col, jnp.full((e_pt * NW - e,), n, jnp.int32)])
    ep = cpt * ch
    pad_rows = jnp.broadcast_to(
        jnp.arange(ep - e_pt, dtype=jnp.int32) % n, (NW, ep - e_pt))
    pad_cols = jnp.broadcast_to(
        n + jnp.arange(ep - e_pt, dtype=jnp.int32) % (n_pad - n),
        (NW, ep - e_pt))
    rowi = jnp.concatenate(
        [row_p.reshape(NW, e_pt), pad_rows], axis=1).reshape(NW, cpt, ch)
    coli = jnp.concatenate(
        [col_p.reshape(NW, e_pt), pad_cols], axis=1).reshape(NW, cpt, ch)
    posx = jnp.pad(pos[:, 0], (0, n_pad - n))
    posy = jnp.pad(pos[:, 1], (0, n_pad - n))
    posz = jnp.pad(pos[:, 2], (0, n_pad - n))
    x_pad = jnp.pad(x, ((0, n_pad - n), (0, 0)))
    zeros_h = jnp.zeros((n_pad, dh), jnp.float32)
    batch2 = jnp.pad(batch, (0, n_pad - n), constant_values=NG).reshape(1, n_pad)
    mu_col = mu.reshape(nrbf, 1)

    d2 = _sc_d2(posx, posy, posz, rowi, coli, n_pad, cpt, ch)
    d2m = d2.reshape(e_pad // 128, 128)

    h_lo, h_hi = _tc_dense_in(x_pad, W_in, b_in.reshape(1, d), n_pad, d, dh)
    for We, be, W, b in ((We1, be1, W1, b1), (We2, be2, W2, b2),
                         (We3, be3, W3, b3)):
        ef = _tc_ef(d2m, mu_col, We, be.reshape(1, d), e_pad, nrbf, d)
        p_lo, p_hi = _sc_conv(h_lo, h_hi, ef.reshape(e_pad, d), rowi, coli,
                              zeros_h, n_pad, cpt, ch, d, dh)
        h_lo, h_hi = _tc_dense_layer(p_lo, p_hi, W, b.reshape(1, d),
                                     n_pad, d, dh)

    return _tc_segmean(h_lo, h_hi, batch2, n_pad, d, dh)


# speedup vs baseline: 5.9093x; 1.0011x over previous
import functools

import jax
import jax.numpy as jnp
from jax import lax
from jax.experimental import pallas as pl
from jax.experimental.pallas import tpu as pltpu
from jax.experimental.pallas import tpu_sc as plsc

NC = 2
NS = 16
LANES = 16
NW = NC * NS

GAMMA = 10.0
NG = 32


def _sc_d2(posx, posy, posz, rowi, coli, n_pad, cpt, ch):
    mesh = plsc.VectorSubcoreMesh(core_axis_name="c", subcore_axis_name="s")

    @functools.partial(
        pl.kernel,
        out_type=jax.ShapeDtypeStruct((NW, cpt, ch), jnp.float32),
        mesh=mesh,
        scratch_types=[
            pltpu.VMEM((n_pad,), jnp.float32),
            pltpu.VMEM((n_pad,), jnp.float32),
            pltpu.VMEM((n_pad,), jnp.float32),
            pltpu.VMEM((cpt, ch), jnp.int32),
            pltpu.VMEM((cpt, ch), jnp.int32),
            pltpu.VMEM((cpt, ch), jnp.float32),
        ],
        compiler_params=pltpu.CompilerParams(needs_layout_passes=False),
    )
    def run(px_hbm, py_hbm, pz_hbm, rowi_hbm, coli_hbm, d2_hbm,
            px, py, pz, row_v, col_v, d2_v):
        c = lax.axis_index("c")
        s = lax.axis_index("s")
        wid = c * NS + s
        pltpu.sync_copy(px_hbm, px)
        pltpu.sync_copy(py_hbm, py)
        pltpu.sync_copy(pz_hbm, pz)
        pltpu.sync_copy(rowi_hbm.at[wid], row_v)
        pltpu.sync_copy(coli_hbm.at[wid], col_v)

        def chunk(j, carry):
            def sub(k, carry2):
                sl = pl.ds(k * LANES, LANES)
                ri = row_v[j, sl]
                ci = col_v[j, sl]
                dx = plsc.load_gather(px, [ri]) - plsc.load_gather(px, [ci])
                dy = plsc.load_gather(py, [ri]) - plsc.load_gather(py, [ci])
                dz = plsc.load_gather(pz, [ri]) - plsc.load_gather(pz, [ci])
                d2_v[j, sl] = dx * dx + dy * dy + dz * dz
                return carry2

            return lax.fori_loop(0, ch // LANES, sub, carry)

        lax.fori_loop(0, cpt, chunk, 0)
        pltpu.sync_copy(d2_v, d2_hbm.at[wid])

    return run(posx, posy, posz, rowi, coli)


def _sc_conv(h_lo, h_hi, ef, rowi, coli, zeros_h, n_pad, cpt, ch, d, dh):
    mesh = plsc.VectorSubcoreMesh(core_axis_name="c", subcore_axis_name="s")
    rps = n_pad // NS
    buf = pltpu.VMEM((ch, dh), jnp.float32)
    part_sd = jax.ShapeDtypeStruct((NC, n_pad, dh), jnp.float32)

    @functools.partial(
        pl.kernel,
        out_type=(part_sd, part_sd),
        mesh=mesh,
        scratch_types=[
            pltpu.VMEM((cpt, ch), jnp.int32),
            pltpu.VMEM((cpt, ch), jnp.int32),
            buf, buf, buf, buf, buf, buf,
            pltpu.VMEM_SHARED((n_pad, dh), jnp.float32),
            pltpu.SemaphoreType.DMA, pltpu.SemaphoreType.DMA,
            pltpu.SemaphoreType.DMA, pltpu.SemaphoreType.DMA,
            pltpu.SemaphoreType.DMA, pltpu.SemaphoreType.DMA,
        ],
        compiler_params=pltpu.CompilerParams(needs_layout_passes=False,
                                             use_tc_tiling_on_sc=False),
    )
    def run(hlo_hbm, hhi_hbm, ef_hbm, rowi_hbm, coli_hbm, z_hbm,
            outlo_hbm, outhi_hbm,
            row_v, col_v, rows_a, rows_b, efa, efb, msga, msgb, aggr,
            gs_a, gs_b, es_a, es_b, ss_a, ss_b):
        c = lax.axis_index("c")
        s = lax.axis_index("s")
        wid = c * NS + s
        mine = pl.ds(s * rps, rps)
        pltpu.sync_copy(rowi_hbm.at[wid], row_v)
        pltpu.sync_copy(coli_hbm.at[wid], col_v)
        slots = ((rows_a, efa, msga, gs_a, es_a, ss_a),
                 (rows_b, efb, msgb, gs_b, es_b, ss_b))

        for hx, (h_hbm, out_hbm) in enumerate(((hlo_hbm, outlo_hbm),
                                               (hhi_hbm, outhi_hbm))):
            off = pl.ds(hx * dh, dh)
            pltpu.sync_copy(z_hbm.at[mine], aggr.at[mine])
            plsc.subcore_barrier()

            def start_fetch(j, rows_v, ef_v, gsem, esem):
                pltpu.async_copy(h_hbm.at[row_v.at[j]], rows_v, gsem)
                pltpu.async_copy(
                    ef_hbm.at[pl.ds((wid * cpt + j) * ch, ch), off],
                    ef_v, esem)

            for b in range(2):
                rows_v, ef_v, _, gsem, esem, _ = slots[b]
                start_fetch(b, rows_v, ef_v, gsem, esem)

            def pair(j2, carry):
                for b in range(2):
                    rows_v, ef_v, msg_v, gsem, esem, ssem = slots[b]
                    j = 2 * j2 + b
                    pltpu.make_async_copy(h_hbm.at[row_v.at[j]], rows_v,
                                          gsem).wait()
                    pltpu.make_async_copy(
                        ef_hbm.at[pl.ds((wid * cpt + j) * ch, ch), off],
                        ef_v, esem).wait()

                    @pl.when(j >= 2)
                    def _():
                        pltpu.make_async_copy(msg_v, aggr.at[col_v.at[j]],
                                              ssem).wait()

                    def rloop(r4, c2):
                        for rr in range(4):
                            r = r4 * 4 + rr
                            for q in range(dh // LANES):
                                sl = pl.ds(q * LANES, LANES)
                                msg_v[r, sl] = rows_v[r, sl] * ef_v[r, sl]
                        return c2

                    lax.fori_loop(0, ch // 4, rloop, 0)

                    @pl.when(j + 2 < cpt)
                    def _():
                        start_fetch(j + 2, rows_v, ef_v, gsem, esem)

                    pltpu.async_copy(msg_v, aggr.at[col_v.at[j]], ssem,
                                     add=True)
                return carry

            lax.fori_loop(0, cpt // 2, pair, 0)
            for b in range(2):
                _, _, msg_v, _, _, ssem = slots[b]
                j = cpt - 2 + b
                pltpu.make_async_copy(msg_v, aggr.at[col_v.at[j]], ssem).wait()
            plsc.subcore_barrier()
            pltpu.sync_copy(aggr.at[mine], out_hbm.at[c, mine])
            plsc.subcore_barrier()

    return run(h_lo, h_hi, ef, rowi, coli, zeros_h)


def _tc_ef(d2m, mu_col, We, be, e_pad, nrbf, d):
    eb = 1024
    sb = eb // 128
    grid = (e_pad // eb,)

    def body(d2_ref, mu_ref, w, b, o):
        cols = []
        for s in range(sb):
            dist = jnp.sqrt(d2_ref[s:s + 1, :] + 1e-12)
            delta = dist - mu_ref[...]
            cols.append(jnp.exp(-GAMMA * (delta * delta)))
        rbf_t = jnp.concatenate(cols, axis=1)
        y = lax.dot_general(rbf_t, w[...], (((0,), (0,)), ((), ())),
                            preferred_element_type=jnp.float32)
        yb = y + b[...]
        for s in range(sb):
            o[s] = yb[s * 128:(s + 1) * 128, :]

    full = lambda shape: pl.BlockSpec(shape, lambda i: (0,) * len(shape))
    return pl.pallas_call(
        body,
        grid=grid,
        in_specs=[
            pl.BlockSpec((sb, 128), lambda i: (i, 0)),
            full((nrbf, 1)),
            full((nrbf, d)),
            full((1, d)),
        ],
        out_specs=pl.BlockSpec((sb, 128, d), lambda i: (i, 0, 0)),
        out_shape=jax.ShapeDtypeStruct((e_pad // 128, 128, d), jnp.float32),
    )(d2m, mu_col, We, be)


def _tc_dense_in(a, W, b, n_pad, d, dh):
    rb = 512
    grid = (n_pad // rb,)

    def body(a_ref, w_ref, b_ref, ol_ref, oh_ref):
        y = (jnp.dot(a_ref[...], w_ref[...], preferred_element_type=jnp.float32)
             + b_ref[...])
        ol_ref[...] = y[:, :dh]
        oh_ref[...] = y[:, dh:]

    out_sd = jax.ShapeDtypeStruct((n_pad, dh), jnp.float32)
    return pl.pallas_call(
        body,
        grid=grid,
        in_specs=[
            pl.BlockSpec((rb, d), lambda i: (i, 0)),
            pl.BlockSpec((d, d), lambda i: (0, 0)),
            pl.BlockSpec((1, d), lambda i: (0, 0)),
        ],
        out_specs=[pl.BlockSpec((rb, dh), lambda i: (i, 0))] * 2,
        out_shape=[out_sd] * 2,
    )(a, W, b)


def _tc_dense_layer(p_lo, p_hi, W, b, n_pad, d, dh):
    rb = 512
    grid = (n_pad // rb,)

    def body(pl_ref, ph_ref, w_ref, b_ref, ol_ref, oh_ref):
        acc = jnp.concatenate(
            [jnp.sum(pl_ref[...], axis=0), jnp.sum(ph_ref[...], axis=0)],
            axis=1)
        y = jnp.dot(acc, w_ref[...], preferred_element_type=jnp.float32) + b_ref[...]
        y = jnp.maximum(y, 0.0)
        ol_ref[...] = y[:, :dh]
        oh_ref[...] = y[:, dh:]

    out_sd = jax.ShapeDtypeStruct((n_pad, dh), jnp.float32)
    return pl.pallas_call(
        body,
        grid=grid,
        in_specs=[
            pl.BlockSpec((NC, rb, dh), lambda i: (0, i, 0)),
            pl.BlockSpec((NC, rb, dh), lambda i: (0, i, 0)),
            pl.BlockSpec((d, d), lambda i: (0, 0)),
            pl.BlockSpec((1, d), lambda i: (0, 0)),
        ],
        out_specs=[pl.BlockSpec((rb, dh), lambda i: (i, 0))] * 2,
        out_shape=[out_sd] * 2,
    )(p_lo, p_hi, W, b)


def _tc_segmean(h_lo, h_hi, batch2, n_pad, d, dh):
    rb = 1024
    grid = (n_pad // rb,)

    def body(b_ref, hl_ref, hh_ref, o_ref, acc, cacc):
        i = pl.program_id(0)
        seg = lax.broadcasted_iota(jnp.int32, (NG, rb), 0)
        m = (b_ref[...] == seg).astype(jnp.float32)
        sl = jnp.dot(m, hl_ref[...], preferred_element_type=jnp.float32)
        sh = jnp.dot(m, hh_ref[...], preferred_element_type=jnp.float32)
        sums = jnp.concatenate([sl, sh], axis=1)
        cnt = jnp.sum(m, axis=1, keepdims=True)

        @pl.when(i == 0)
        def _():
            acc[...] = jnp.zeros_like(acc)
            cacc[...] = jnp.zeros_like(cacc)

        acc[...] = acc[...] + sums
        cacc[...] = cacc[...] + cnt

        @pl.when(i == grid[0] - 1)
        def _():
            o_ref[...] = acc[...] / jnp.maximum(cacc[...], 1.0)

    return pl.pallas_call(
        body,
        grid=grid,
        in_specs=[
            pl.BlockSpec((1, rb), lambda i: (0, i)),
            pl.BlockSpec((rb, dh), lambda i: (i, 0)),
            pl.BlockSpec((rb, dh), lambda i: (i, 0)),
        ],
        out_specs=pl.BlockSpec((NG, d), lambda i: (0, 0)),
        out_shape=jax.ShapeDtypeStruct((NG, d), jnp.float32),
        scratch_shapes=[
            pltpu.VMEM((NG, d), jnp.float32),
            pltpu.VMEM((NG, 1), jnp.float32),
        ],
    )(batch2, h_lo, h_hi)


def kernel(x, pos, edge_index, edge_attr, batch, W_in, b_in, mu,
           We1, be1, W1, b1, We2, be2, W2, b2, We3, be3, W3, b3):
    n, d = x.shape
    e = edge_index.shape[1]
    nrbf = mu.shape[0]
    dh = d // 2

    ch = 128
    cpt = -(-e // (NW * ch))
    cpt += cpt % 2
    e_pad = NW * cpt * ch
    n_pad = -(-(n + 1) // 2048) * 2048

    row = edge_index[0]
    col = edge_index[1]
    e_pt = -(-e // NW)
    row_p = jnp.concatenate([row, jnp.zeros((e_pt * NW - e,), jnp.int32)])
    col_p = jnp.concatenate([col, jnp.full((e_pt * NW - e,), n, jnp.int32)])
    ep = cpt * ch
    pad_rows = jnp.broadcast_to(
        jnp.arange(ep - e_pt, dtype=jnp.int32) % n, (NW, ep - e_pt))
    pad_cols = jnp.broadcast_to(
        n + jnp.arange(ep - e_pt, dtype=jnp.int32) % (n_pad - n),
        (NW, ep - e_pt))
    rowi = jnp.concatenate(
        [row_p.reshape(NW, e_pt), pad_rows], axis=1).reshape(NW, cpt, ch)
    coli = jnp.concatenate(
        [col_p.reshape(NW, e_pt), pad_cols], axis=1).reshape(NW, cpt, ch)
    posx = jnp.pad(pos[:, 0], (0, n_pad - n))
    posy = jnp.pad(pos[:, 1], (0, n_pad - n))
    posz = jnp.pad(pos[:, 2], (0, n_pad - n))
    x_pad = jnp.pad(x, ((0, n_pad - n), (0, 0)))
    zeros_h = jnp.zeros((n_pad, dh), jnp.float32)
    batch2 = jnp.pad(batch, (0, n_pad - n), constant_values=NG).reshape(1, n_pad)
    mu_col = mu.reshape(nrbf, 1)

    d2 = _sc_d2(posx, posy, posz, rowi, coli, n_pad, cpt, ch)
    d2m = d2.reshape(e_pad // 128, 128)

    h_lo, h_hi = _tc_dense_in(x_pad, W_in, b_in.reshape(1, d), n_pad, d, dh)
    for We, be, W, b in ((We1, be1, W1, b1), (We2, be2, W2, b2),
                         (We3, be3, W3, b3)):
        ef = _tc_ef(d2m, mu_col, We, be.reshape(1, d), e_pad, nrbf, d)
        p_lo, p_hi = _sc_conv(h_lo, h_hi, ef.reshape(e_pad, d), rowi, coli,
                              zeros_h, n_pad, cpt, ch, d, dh)
        h_lo, h_hi = _tc_dense_layer(p_lo, p_hi, W, b.reshape(1, d),
                                     n_pad, d, dh)

    return _tc_segmean(h_lo, h_hi, batch2, n_pad, d, dh)


# speedup vs baseline: 6.8313x; 1.1560x over previous
import functools

import jax
import jax.numpy as jnp
from jax import lax
from jax.experimental import pallas as pl
from jax.experimental.pallas import tpu as pltpu
from jax.experimental.pallas import tpu_sc as plsc

NC = 2
NS = 16
LANES = 16
NW = NC * NS

GAMMA = 10.0
NG = 32


def _sc_d2(posx, posy, posz, rowi, coli, n_pad, cpt, ch):
    mesh = plsc.VectorSubcoreMesh(core_axis_name="c", subcore_axis_name="s")

    @functools.partial(
        pl.kernel,
        out_type=jax.ShapeDtypeStruct((NW, cpt, ch), jnp.float32),
        mesh=mesh,
        scratch_types=[
            pltpu.VMEM((n_pad,), jnp.float32),
            pltpu.VMEM((n_pad,), jnp.float32),
            pltpu.VMEM((n_pad,), jnp.float32),
            pltpu.VMEM((cpt, ch), jnp.int32),
            pltpu.VMEM((cpt, ch), jnp.int32),
            pltpu.VMEM((cpt, ch), jnp.float32),
        ],
        compiler_params=pltpu.CompilerParams(needs_layout_passes=False),
    )
    def run(px_hbm, py_hbm, pz_hbm, rowi_hbm, coli_hbm, d2_hbm,
            px, py, pz, row_v, col_v, d2_v):
        c = lax.axis_index("c")
        s = lax.axis_index("s")
        wid = c * NS + s
        pltpu.sync_copy(px_hbm, px)
        pltpu.sync_copy(py_hbm, py)
        pltpu.sync_copy(pz_hbm, pz)
        pltpu.sync_copy(rowi_hbm.at[wid], row_v)
        pltpu.sync_copy(coli_hbm.at[wid], col_v)

        def chunk(j, carry):
            def sub(k, carry2):
                sl = pl.ds(k * LANES, LANES)
                ri = row_v[j, sl]
                ci = col_v[j, sl]
                dx = plsc.load_gather(px, [ri]) - plsc.load_gather(px, [ci])
                dy = plsc.load_gather(py, [ri]) - plsc.load_gather(py, [ci])
                dz = plsc.load_gather(pz, [ri]) - plsc.load_gather(pz, [ci])
                d2_v[j, sl] = dx * dx + dy * dy + dz * dz
                return carry2

            return lax.fori_loop(0, ch // LANES, sub, carry)

        lax.fori_loop(0, cpt, chunk, 0)
        pltpu.sync_copy(d2_v, d2_hbm.at[wid])

    return run(posx, posy, posz, rowi, coli)


def _sc_conv(h_lo, h_hi, ef, rowi, coli, zeros_h, n_pad, cpt, ch, d, dh):
    mesh = plsc.VectorSubcoreMesh(core_axis_name="c", subcore_axis_name="s")
    rps = n_pad // NS
    buf = pltpu.VMEM((ch, dh), jnp.float32)
    part_sd = jax.ShapeDtypeStruct((NC, n_pad, dh), jnp.float32)

    @functools.partial(
        pl.kernel,
        out_type=(part_sd, part_sd),
        mesh=mesh,
        scratch_types=[
            pltpu.VMEM((cpt, ch), jnp.int32),
            pltpu.VMEM((cpt, ch), jnp.int32),
            buf, buf, buf, buf, buf, buf,
            pltpu.VMEM_SHARED((n_pad, dh), jnp.float32),
            pltpu.SemaphoreType.DMA, pltpu.SemaphoreType.DMA,
            pltpu.SemaphoreType.DMA, pltpu.SemaphoreType.DMA,
            pltpu.SemaphoreType.DMA, pltpu.SemaphoreType.DMA,
        ],
        compiler_params=pltpu.CompilerParams(needs_layout_passes=False,
                                             use_tc_tiling_on_sc=False),
    )
    def run(hlo_hbm, hhi_hbm, ef_hbm, rowi_hbm, coli_hbm, z_hbm,
            outlo_hbm, outhi_hbm,
            row_v, col_v, rows_a, rows_b, efa, efb, msga, msgb, aggr,
            gs_a, gs_b, es_a, es_b, ss_a, ss_b):
        c = lax.axis_index("c")
        s = lax.axis_index("s")
        wid = c * NS + s
        mine = pl.ds(s * rps, rps)
        pltpu.sync_copy(rowi_hbm.at[wid], row_v)
        pltpu.sync_copy(coli_hbm.at[wid], col_v)
        slots = ((rows_a, efa, msga, gs_a, es_a, ss_a),
                 (rows_b, efb, msgb, gs_b, es_b, ss_b))

        for hx, (h_hbm, out_hbm) in enumerate(((hlo_hbm, outlo_hbm),
                                               (hhi_hbm, outhi_hbm))):
            off = pl.ds(hx * dh, dh)
            pltpu.sync_copy(z_hbm.at[mine], aggr.at[mine])
            plsc.subcore_barrier()

            def start_fetch(j, rows_v, ef_v, gsem, esem):
                pltpu.async_copy(h_hbm.at[row_v.at[j]], rows_v, gsem)
                pltpu.async_copy(
                    ef_hbm.at[pl.ds((wid * cpt + j) * ch, ch), off],
                    ef_v, esem)

            for b in range(2):
                rows_v, ef_v, _, gsem, esem, _ = slots[b]
                start_fetch(b, rows_v, ef_v, gsem, esem)

            def pair(j2, carry):
                for b in range(2):
                    rows_v, ef_v, msg_v, gsem, esem, ssem = slots[b]
                    j = 2 * j2 + b
                    pltpu.make_async_copy(h_hbm.at[row_v.at[j]], rows_v,
                                          gsem).wait()
                    pltpu.make_async_copy(
                        ef_hbm.at[pl.ds((wid * cpt + j) * ch, ch), off],
                        ef_v, esem).wait()

                    @pl.when(j >= 2)
                    def _():
                        pltpu.make_async_copy(msg_v, aggr.at[col_v.at[j]],
                                              ssem).wait()

                    def rloop(r4, c2):
                        for rr in range(4):
                            r = r4 * 4 + rr
                            for q in range(dh // LANES):
                                sl = pl.ds(q * LANES, LANES)
                                msg_v[r, sl] = rows_v[r, sl] * ef_v[r, sl]
                        return c2

                    lax.fori_loop(0, ch // 4, rloop, 0)

                    @pl.when(j + 2 < cpt)
                    def _():
                        start_fetch(j + 2, rows_v, ef_v, gsem, esem)

                    pltpu.async_copy(msg_v, aggr.at[col_v.at[j]], ssem,
                                     add=True)
                return carry

            lax.fori_loop(0, cpt // 2, pair, 0)
            for b in range(2):
                _, _, msg_v, _, _, ssem = slots[b]
                j = cpt - 2 + b
                pltpu.make_async_copy(msg_v, aggr.at[col_v.at[j]], ssem).wait()
            plsc.subcore_barrier()
            pltpu.sync_copy(aggr.at[mine], out_hbm.at[c, mine])
            plsc.subcore_barrier()

    return run(h_lo, h_hi, ef, rowi, coli, zeros_h)


def _tc_ef(d2m, mu_col, We, be, e_pad, nrbf, d):
    eb = 4096
    sb = eb // 128
    grid = (e_pad // eb,)

    def body(d2_ref, mu_ref, w, b, o):
        cols = []
        for s in range(sb):
            dist = jnp.sqrt(d2_ref[s:s + 1, :] + 1e-12)
            delta = dist - mu_ref[...]
            cols.append(jnp.exp(-GAMMA * (delta * delta)))
        rbf_t = jnp.concatenate(cols, axis=1)
        y = lax.dot_general(rbf_t, w[...], (((0,), (0,)), ((), ())),
                            preferred_element_type=jnp.float32)
        yb = y + b[...]
        for s in range(sb):
            o[s] = yb[s * 128:(s + 1) * 128, :]

    full = lambda shape: pl.BlockSpec(shape, lambda i: (0,) * len(shape))
    return pl.pallas_call(
        body,
        grid=grid,
        in_specs=[
            pl.BlockSpec((sb, 128), lambda i: (i, 0)),
            full((nrbf, 1)),
            full((nrbf, d)),
            full((1, d)),
        ],
        out_specs=pl.BlockSpec((sb, 128, d), lambda i: (i, 0, 0)),
        out_shape=jax.ShapeDtypeStruct((e_pad // 128, 128, d), jnp.float32),
    )(d2m, mu_col, We, be)


def _tc_dense_in(a, W, b, n_pad, d, dh):
    rb = 512
    grid = (n_pad // rb,)

    def body(a_ref, w_ref, b_ref, ol_ref, oh_ref):
        y = (jnp.dot(a_ref[...], w_ref[...], preferred_element_type=jnp.float32)
             + b_ref[...])
        ol_ref[...] = y[:, :dh]
        oh_ref[...] = y[:, dh:]

    out_sd = jax.ShapeDtypeStruct((n_pad, dh), jnp.float32)
    return pl.pallas_call(
        body,
        grid=grid,
        in_specs=[
            pl.BlockSpec((rb, d), lambda i: (i, 0)),
            pl.BlockSpec((d, d), lambda i: (0, 0)),
            pl.BlockSpec((1, d), lambda i: (0, 0)),
        ],
        out_specs=[pl.BlockSpec((rb, dh), lambda i: (i, 0))] * 2,
        out_shape=[out_sd] * 2,
    )(a, W, b)


def _tc_dense_layer(p_lo, p_hi, W, b, n_pad, d, dh):
    rb = 512
    grid = (n_pad // rb,)

    def body(pl_ref, ph_ref, w_ref, b_ref, ol_ref, oh_ref):
        acc = jnp.concatenate(
            [jnp.sum(pl_ref[...], axis=0), jnp.sum(ph_ref[...], axis=0)],
            axis=1)
        y = jnp.dot(acc, w_ref[...], preferred_element_type=jnp.float32) + b_ref[...]
        y = jnp.maximum(y, 0.0)
        ol_ref[...] = y[:, :dh]
        oh_ref[...] = y[:, dh:]

    out_sd = jax.ShapeDtypeStruct((n_pad, dh), jnp.float32)
    return pl.pallas_call(
        body,
        grid=grid,
        in_specs=[
            pl.BlockSpec((NC, rb, dh), lambda i: (0, i, 0)),
            pl.BlockSpec((NC, rb, dh), lambda i: (0, i, 0)),
            pl.BlockSpec((d, d), lambda i: (0, 0)),
            pl.BlockSpec((1, d), lambda i: (0, 0)),
        ],
        out_specs=[pl.BlockSpec((rb, dh), lambda i: (i, 0))] * 2,
        out_shape=[out_sd] * 2,
    )(p_lo, p_hi, W, b)


def _tc_segmean(h_lo, h_hi, batch2, n_pad, d, dh):
    rb = 1024
    grid = (n_pad // rb,)

    def body(b_ref, hl_ref, hh_ref, o_ref, acc, cacc):
        i = pl.program_id(0)
        seg = lax.broadcasted_iota(jnp.int32, (NG, rb), 0)
        m = (b_ref[...] == seg).astype(jnp.float32)
        sl = jnp.dot(m, hl_ref[...], preferred_element_type=jnp.float32)
        sh = jnp.dot(m, hh_ref[...], preferred_element_type=jnp.float32)
        sums = jnp.concatenate([sl, sh], axis=1)
        cnt = jnp.sum(m, axis=1, keepdims=True)

        @pl.when(i == 0)
        def _():
            acc[...] = jnp.zeros_like(acc)
            cacc[...] = jnp.zeros_like(cacc)

        acc[...] = acc[...] + sums
        cacc[...] = cacc[...] + cnt

        @pl.when(i == grid[0] - 1)
        def _():
            o_ref[...] = acc[...] / jnp.maximum(cacc[...], 1.0)

    return pl.pallas_call(
        body,
        grid=grid,
        in_specs=[
            pl.BlockSpec((1, rb), lambda i: (0, i)),
            pl.BlockSpec((rb, dh), lambda i: (i, 0)),
            pl.BlockSpec((rb, dh), lambda i: (i, 0)),
        ],
        out_specs=pl.BlockSpec((NG, d), lambda i: (0, 0)),
        out_shape=jax.ShapeDtypeStruct((NG, d), jnp.float32),
        scratch_shapes=[
            pltpu.VMEM((NG, d), jnp.float32),
            pltpu.VMEM((NG, 1), jnp.float32),
        ],
    )(batch2, h_lo, h_hi)


def kernel(x, pos, edge_index, edge_attr, batch, W_in, b_in, mu,
           We1, be1, W1, b1, We2, be2, W2, b2, We3, be3, W3, b3):
    n, d = x.shape
    e = edge_index.shape[1]
    nrbf = mu.shape[0]
    dh = d // 2

    ch = 128
    cpt = -(-e // (NW * ch))
    cpt += cpt % 2
    e_pad = NW * cpt * ch
    n_pad = -(-(n + 1) // 2048) * 2048

    row = edge_index[0]
    col = edge_index[1]
    e_pt = -(-e // NW)
    row_p = jnp.concatenate([row, jnp.zeros((e_pt * NW - e,), jnp.int32)])
    col_p = jnp.concatenate([col, jnp.full((e_pt * NW - e,), n, jnp.int32)])
    ep = cpt * ch
    pad_rows = jnp.broadcast_to(
        jnp.arange(ep - e_pt, dtype=jnp.int32) % n, (NW, ep - e_pt))
    pad_cols = jnp.broadcast_to(
        n + jnp.arange(ep - e_pt, dtype=jnp.int32) % (n_pad - n),
        (NW, ep - e_pt))
    rowi = jnp.concatenate(
        [row_p.reshape(NW, e_pt), pad_rows], axis=1).reshape(NW, cpt, ch)
    coli = jnp.concatenate(
        [col_p.reshape(NW, e_pt), pad_cols], axis=1).reshape(NW, cpt, ch)
    posx = jnp.pad(pos[:, 0], (0, n_pad - n))
    posy = jnp.pad(pos[:, 1], (0, n_pad - n))
    posz = jnp.pad(pos[:, 2], (0, n_pad - n))
    x_pad = jnp.pad(x, ((0, n_pad - n), (0, 0)))
    zeros_h = jnp.zeros((n_pad, dh), jnp.float32)
    batch2 = jnp.pad(batch, (0, n_pad - n), constant_values=NG).reshape(1, n_pad)
    mu_col = mu.reshape(nrbf, 1)

    d2 = _sc_d2(posx, posy, posz, rowi, coli, n_pad, cpt, ch)
    d2m = d2.reshape(e_pad // 128, 128)

    h_lo, h_hi = _tc_dense_in(x_pad, W_in, b_in.reshape(1, d), n_pad, d, dh)
    for We, be, W, b in ((We1, be1, W1, b1), (We2, be2, W2, b2),
                         (We3, be3, W3, b3)):
        ef = _tc_ef(d2m, mu_col, We, be.reshape(1, d), e_pad, nrbf, d)
        p_lo, p_hi = _sc_conv(h_lo, h_hi, ef.reshape(e_pad, d), rowi, coli,
                              zeros_h, n_pad, cpt, ch, d, dh)
        h_lo, h_hi = _tc_dense_layer(p_lo, p_hi, W, b.reshape(1, d),
                                     n_pad, d, dh)

    return _tc_segmean(h_lo, h_hi, batch2, n_pad, d, dh)


# speedup vs baseline: 6.9287x; 1.0143x over previous
import functools

import jax
import jax.numpy as jnp
from jax import lax
from jax.experimental import pallas as pl
from jax.experimental.pallas import tpu as pltpu
from jax.experimental.pallas import tpu_sc as plsc

NC = 2
NS = 16
LANES = 16
NW = NC * NS

GAMMA = 10.0
NG = 32


def _sc_d2(posx, posy, posz, rowi, coli, n_pad, cpt, ch):
    mesh = plsc.VectorSubcoreMesh(core_axis_name="c", subcore_axis_name="s")

    @functools.partial(
        pl.kernel,
        out_type=jax.ShapeDtypeStruct((NW, cpt, ch), jnp.float32),
        mesh=mesh,
        scratch_types=[
            pltpu.VMEM((n_pad,), jnp.float32),
            pltpu.VMEM((n_pad,), jnp.float32),
            pltpu.VMEM((n_pad,), jnp.float32),
            pltpu.VMEM((cpt, ch), jnp.int32),
            pltpu.VMEM((cpt, ch), jnp.int32),
            pltpu.VMEM((cpt, ch), jnp.float32),
        ],
        compiler_params=pltpu.CompilerParams(needs_layout_passes=False),
    )
    def run(px_hbm, py_hbm, pz_hbm, rowi_hbm, coli_hbm, d2_hbm,
            px, py, pz, row_v, col_v, d2_v):
        c = lax.axis_index("c")
        s = lax.axis_index("s")
        wid = c * NS + s
        pltpu.sync_copy(px_hbm, px)
        pltpu.sync_copy(py_hbm, py)
        pltpu.sync_copy(pz_hbm, pz)
        pltpu.sync_copy(rowi_hbm.at[wid], row_v)
        pltpu.sync_copy(coli_hbm.at[wid], col_v)

        def chunk(j, carry):
            def sub(k, carry2):
                sl = pl.ds(k * LANES, LANES)
                ri = row_v[j, sl]
                ci = col_v[j, sl]
                dx = plsc.load_gather(px, [ri]) - plsc.load_gather(px, [ci])
                dy = plsc.load_gather(py, [ri]) - plsc.load_gather(py, [ci])
                dz = plsc.load_gather(pz, [ri]) - plsc.load_gather(pz, [ci])
                d2_v[j, sl] = dx * dx + dy * dy + dz * dz
                return carry2

            return lax.fori_loop(0, ch // LANES, sub, carry)

        lax.fori_loop(0, cpt, chunk, 0)
        pltpu.sync_copy(d2_v, d2_hbm.at[wid])

    return run(posx, posy, posz, rowi, coli)


def _sc_conv(h_lo, h_hi, ef, rowi, coli, zeros_h, n_pad, cpt, ch, d, dh):
    mesh = plsc.VectorSubcoreMesh(core_axis_name="c", subcore_axis_name="s")
    rps = n_pad // NS
    buf = pltpu.VMEM((ch, dh), jnp.float32)
    part_sd = jax.ShapeDtypeStruct((NC, n_pad, dh), jnp.float32)

    @functools.partial(
        pl.kernel,
        out_type=(part_sd, part_sd),
        mesh=mesh,
        scratch_types=[
            pltpu.VMEM((cpt, ch), jnp.int32),
            pltpu.VMEM((cpt, ch), jnp.int32),
            buf, buf, buf, buf, buf, buf,
            pltpu.VMEM_SHARED((n_pad, dh), jnp.float32),
            pltpu.SemaphoreType.DMA, pltpu.SemaphoreType.DMA,
            pltpu.SemaphoreType.DMA, pltpu.SemaphoreType.DMA,
            pltpu.SemaphoreType.DMA, pltpu.SemaphoreType.DMA,
        ],
        compiler_params=pltpu.CompilerParams(needs_layout_passes=False,
                                             use_tc_tiling_on_sc=False),
    )
    def run(hlo_hbm, hhi_hbm, ef_hbm, rowi_hbm, coli_hbm, z_hbm,
            outlo_hbm, outhi_hbm,
            row_v, col_v, rows_a, rows_b, efa, efb, msga, msgb, aggr,
            gs_a, gs_b, es_a, es_b, ss_a, ss_b):
        c = lax.axis_index("c")
        s = lax.axis_index("s")
        wid = c * NS + s
        mine = pl.ds(s * rps, rps)
        pltpu.sync_copy(rowi_hbm.at[wid], row_v)
        pltpu.sync_copy(coli_hbm.at[wid], col_v)
        slots = ((rows_a, efa, msga, gs_a, es_a, ss_a),
                 (rows_b, efb, msgb, gs_b, es_b, ss_b))

        for hx, (h_hbm, out_hbm) in enumerate(((hlo_hbm, outlo_hbm),
                                               (hhi_hbm, outhi_hbm))):
            off = pl.ds(hx * dh, dh)
            pltpu.sync_copy(z_hbm.at[mine], aggr.at[mine])
            plsc.subcore_barrier()

            def start_fetch(j, rows_v, ef_v, gsem, esem):
                pltpu.async_copy(h_hbm.at[row_v.at[j]], rows_v, gsem)
                pltpu.async_copy(
                    ef_hbm.at[pl.ds((wid * cpt + j) * ch, ch), off],
                    ef_v, esem)

            for b in range(2):
                rows_v, ef_v, _, gsem, esem, _ = slots[b]
                start_fetch(b, rows_v, ef_v, gsem, esem)

            def pair(j2, carry):
                for b in range(2):
                    rows_v, ef_v, msg_v, gsem, esem, ssem = slots[b]
                    j = 2 * j2 + b
                    pltpu.make_async_copy(h_hbm.at[row_v.at[j]], rows_v,
                                          gsem).wait()
                    pltpu.make_async_copy(
                        ef_hbm.at[pl.ds((wid * cpt + j) * ch, ch), off],
                        ef_v, esem).wait()

                    @pl.when(j >= 2)
                    def _():
                        pltpu.make_async_copy(msg_v, aggr.at[col_v.at[j]],
                                              ssem).wait()

                    def rloop(r4, c2):
                        for rr in range(4):
                            r = r4 * 4 + rr
                            for q in range(dh // LANES):
                                sl = pl.ds(q * LANES, LANES)
                                msg_v[r, sl] = rows_v[r, sl] * ef_v[r, sl]
                        return c2

                    lax.fori_loop(0, ch // 4, rloop, 0)

                    @pl.when(j + 2 < cpt)
                    def _():
                        start_fetch(j + 2, rows_v, ef_v, gsem, esem)

                    pltpu.async_copy(msg_v, aggr.at[col_v.at[j]], ssem,
                                     add=True)
                return carry

            lax.fori_loop(0, cpt // 2, pair, 0)
            for b in range(2):
                _, _, msg_v, _, _, ssem = slots[b]
                j = cpt - 2 + b
                pltpu.make_async_copy(msg_v, aggr.at[col_v.at[j]], ssem).wait()
            plsc.subcore_barrier()
            pltpu.sync_copy(aggr.at[mine], out_hbm.at[c, mine])
            plsc.subcore_barrier()

    return run(h_lo, h_hi, ef, rowi, coli, zeros_h)


def _tc_ef(d2m, mu_col, We, be, e_pad, nrbf, d):
    eb = 8192
    sb = eb // 128
    grid = (e_pad // eb,)

    def body(d2_ref, mu_ref, w, b, o):
        cols = []
        for s in range(sb):
            dist = jnp.sqrt(d2_ref[s:s + 1, :] + 1e-12)
            delta = dist - mu_ref[...]
            cols.append(jnp.exp(-GAMMA * (delta * delta)))
        rbf_t = jnp.concatenate(cols, axis=1)
        y = lax.dot_general(rbf_t, w[...], (((0,), (0,)), ((), ())),
                            preferred_element_type=jnp.float32)
        yb = y + b[...]
        for s in range(sb):
            o[s] = yb[s * 128:(s + 1) * 128, :]

    full = lambda shape: pl.BlockSpec(shape, lambda i: (0,) * len(shape))
    return pl.pallas_call(
        body,
        grid=grid,
        in_specs=[
            pl.BlockSpec((sb, 128), lambda i: (i, 0)),
            full((nrbf, 1)),
            full((nrbf, d)),
            full((1, d)),
        ],
        out_specs=pl.BlockSpec((sb, 128, d), lambda i: (i, 0, 0)),
        out_shape=jax.ShapeDtypeStruct((e_pad // 128, 128, d), jnp.float32),
    )(d2m, mu_col, We, be)


def _tc_dense_in(a, W, b, n_pad, d, dh):
    rb = 512
    grid = (n_pad // rb,)

    def body(a_ref, w_ref, b_ref, ol_ref, oh_ref):
        y = (jnp.dot(a_ref[...], w_ref[...], preferred_element_type=jnp.float32)
             + b_ref[...])
        ol_ref[...] = y[:, :dh]
        oh_ref[...] = y[:, dh:]

    out_sd = jax.ShapeDtypeStruct((n_pad, dh), jnp.float32)
    return pl.pallas_call(
        body,
        grid=grid,
        in_specs=[
            pl.BlockSpec((rb, d), lambda i: (i, 0)),
            pl.BlockSpec((d, d), lambda i: (0, 0)),
            pl.BlockSpec((1, d), lambda i: (0, 0)),
        ],
        out_specs=[pl.BlockSpec((rb, dh), lambda i: (i, 0))] * 2,
        out_shape=[out_sd] * 2,
    )(a, W, b)


def _tc_dense_layer(p_lo, p_hi, W, b, n_pad, d, dh):
    rb = 512
    grid = (n_pad // rb,)

    def body(pl_ref, ph_ref, w_ref, b_ref, ol_ref, oh_ref):
        acc = jnp.concatenate(
            [jnp.sum(pl_ref[...], axis=0), jnp.sum(ph_ref[...], axis=0)],
            axis=1)
        y = jnp.dot(acc, w_ref[...], preferred_element_type=jnp.float32) + b_ref[...]
        y = jnp.maximum(y, 0.0)
        ol_ref[...] = y[:, :dh]
        oh_ref[...] = y[:, dh:]

    out_sd = jax.ShapeDtypeStruct((n_pad, dh), jnp.float32)
    return pl.pallas_call(
        body,
        grid=grid,
        in_specs=[
            pl.BlockSpec((NC, rb, dh), lambda i: (0, i, 0)),
            pl.BlockSpec((NC, rb, dh), lambda i: (0, i, 0)),
            pl.BlockSpec((d, d), lambda i: (0, 0)),
            pl.BlockSpec((1, d), lambda i: (0, 0)),
        ],
        out_specs=[pl.BlockSpec((rb, dh), lambda i: (i, 0))] * 2,
        out_shape=[out_sd] * 2,
    )(p_lo, p_hi, W, b)


def _tc_segmean(h_lo, h_hi, batch2, n_pad, d, dh):
    rb = 1024
    grid = (n_pad // rb,)

    def body(b_ref, hl_ref, hh_ref, o_ref, acc, cacc):
        i = pl.program_id(0)
        seg = lax.broadcasted_iota(jnp.int32, (NG, rb), 0)
        m = (b_ref[...] == seg).astype(jnp.float32)
        sl = jnp.dot(m, hl_ref[...], preferred_element_type=jnp.float32)
        sh = jnp.dot(m, hh_ref[...], preferred_element_type=jnp.float32)
        sums = jnp.concatenate([sl, sh], axis=1)
        cnt = jnp.sum(m, axis=1, keepdims=True)

        @pl.when(i == 0)
        def _():
            acc[...] = jnp.zeros_like(acc)
            cacc[...] = jnp.zeros_like(cacc)

        acc[...] = acc[...] + sums
        cacc[...] = cacc[...] + cnt

        @pl.when(i == grid[0] - 1)
        def _():
            o_ref[...] = acc[...] / jnp.maximum(cacc[...], 1.0)

    return pl.pallas_call(
        body,
        grid=grid,
        in_specs=[
            pl.BlockSpec((1, rb), lambda i: (0, i)),
            pl.BlockSpec((rb, dh), lambda i: (i, 0)),
            pl.BlockSpec((rb, dh), lambda i: (i, 0)),
        ],
        out_specs=pl.BlockSpec((NG, d), lambda i: (0, 0)),
        out_shape=jax.ShapeDtypeStruct((NG, d), jnp.float32),
        scratch_shapes=[
            pltpu.VMEM((NG, d), jnp.float32),
            pltpu.VMEM((NG, 1), jnp.float32),
        ],
    )(batch2, h_lo, h_hi)


def kernel(x, pos, edge_index, edge_attr, batch, W_in, b_in, mu,
           We1, be1, W1, b1, We2, be2, W2, b2, We3, be3, W3, b3):
    n, d = x.shape
    e = edge_index.shape[1]
    nrbf = mu.shape[0]
    dh = d // 2

    ch = 128
    cpt = -(-e // (NW * ch))
    cpt += cpt % 2
    e_pad = NW * cpt * ch
    n_pad = -(-(n + 1) // 2048) * 2048

    row = edge_index[0]
    col = edge_index[1]
    e_pt = -(-e // NW)
    row_p = jnp.concatenate([row, jnp.zeros((e_pt * NW - e,), jnp.int32)])
    col_p = jnp.concatenate([col, jnp.full((e_pt * NW - e,), n, jnp.int32)])
    ep = cpt * ch
    pad_rows = jnp.broadcast_to(
        jnp.arange(ep - e_pt, dtype=jnp.int32) % n, (NW, ep - e_pt))
    pad_cols = jnp.broadcast_to(
        n + jnp.arange(ep - e_pt, dtype=jnp.int32) % (n_pad - n),
        (NW, ep - e_pt))
    rowi = jnp.concatenate(
        [row_p.reshape(NW, e_pt), pad_rows], axis=1).reshape(NW, cpt, ch)
    coli = jnp.concatenate(
        [col_p.reshape(NW, e_pt), pad_cols], axis=1).reshape(NW, cpt, ch)
    posx = jnp.pad(pos[:, 0], (0, n_pad - n))
    posy = jnp.pad(pos[:, 1], (0, n_pad - n))
    posz = jnp.pad(pos[:, 2], (0, n_pad - n))
    x_pad = jnp.pad(x, ((0, n_pad - n), (0, 0)))
    zeros_h = jnp.zeros((n_pad, dh), jnp.float32)
    batch2 = jnp.pad(batch, (0, n_pad - n), constant_values=NG).reshape(1, n_pad)
    mu_col = mu.reshape(nrbf, 1)

    d2 = _sc_d2(posx, posy, posz, rowi, coli, n_pad, cpt, ch)
    d2m = d2.reshape(e_pad // 128, 128)

    h_lo, h_hi = _tc_dense_in(x_pad, W_in, b_in.reshape(1, d), n_pad, d, dh)
    for We, be, W, b in ((We1, be1, W1, b1), (We2, be2, W2, b2),
                         (We3, be3, W3, b3)):
        ef = _tc_ef(d2m, mu_col, We, be.reshape(1, d), e_pad, nrbf, d)
        p_lo, p_hi = _sc_conv(h_lo, h_hi, ef.reshape(e_pad, d), rowi, coli,
                              zeros_h, n_pad, cpt, ch, d, dh)
        h_lo, h_hi = _tc_dense_layer(p_lo, p_hi, W, b.reshape(1, d),
                                     n_pad, d, dh)

    return _tc_segmean(h_lo, h_hi, batch2, n_pad, d, dh)


# speedup vs baseline: 7.0708x; 1.0205x over previous
import functools

import jax
import jax.numpy as jnp
from jax import lax
from jax.experimental import pallas as pl
from jax.experimental.pallas import tpu as pltpu
from jax.experimental.pallas import tpu_sc as plsc

NC = 2
NS = 16
LANES = 16
NW = NC * NS

GAMMA = 10.0
NG = 32


def _sc_d2(posx, posy, posz, rowi, coli, n_pad, cpt, ch):
    mesh = plsc.VectorSubcoreMesh(core_axis_name="c", subcore_axis_name="s")

    @functools.partial(
        pl.kernel,
        out_type=jax.ShapeDtypeStruct((NW, cpt, ch), jnp.float32),
        mesh=mesh,
        scratch_types=[
            pltpu.VMEM((n_pad,), jnp.float32),
            pltpu.VMEM((n_pad,), jnp.float32),
            pltpu.VMEM((n_pad,), jnp.float32),
            pltpu.VMEM((cpt, ch), jnp.int32),
            pltpu.VMEM((cpt, ch), jnp.int32),
            pltpu.VMEM((cpt, ch), jnp.float32),
        ],
        compiler_params=pltpu.CompilerParams(needs_layout_passes=False),
    )
    def run(px_hbm, py_hbm, pz_hbm, rowi_hbm, coli_hbm, d2_hbm,
            px, py, pz, row_v, col_v, d2_v):
        c = lax.axis_index("c")
        s = lax.axis_index("s")
        wid = c * NS + s
        pltpu.sync_copy(px_hbm, px)
        pltpu.sync_copy(py_hbm, py)
        pltpu.sync_copy(pz_hbm, pz)
        pltpu.sync_copy(rowi_hbm.at[wid], row_v)
        pltpu.sync_copy(coli_hbm.at[wid], col_v)

        def chunk(j, carry):
            def sub(k, carry2):
                sl = pl.ds(k * LANES, LANES)
                ri = row_v[j, sl]
                ci = col_v[j, sl]
                dx = plsc.load_gather(px, [ri]) - plsc.load_gather(px, [ci])
                dy = plsc.load_gather(py, [ri]) - plsc.load_gather(py, [ci])
                dz = plsc.load_gather(pz, [ri]) - plsc.load_gather(pz, [ci])
                d2_v[j, sl] = dx * dx + dy * dy + dz * dz
                return carry2

            return lax.fori_loop(0, ch // LANES, sub, carry)

        lax.fori_loop(0, cpt, chunk, 0)
        pltpu.sync_copy(d2_v, d2_hbm.at[wid])

    return run(posx, posy, posz, rowi, coli)


def _sc_conv(h_lo, h_hi, ef, rowi, coli, zeros_h, n_pad, cpt, ch, d, dh):
    mesh = plsc.VectorSubcoreMesh(core_axis_name="c", subcore_axis_name="s")
    rps = n_pad // NS
    buf = pltpu.VMEM((ch, dh), jnp.float32)
    part_sd = jax.ShapeDtypeStruct((NC, n_pad, dh), jnp.float32)
    NF = 3
    NM = 2

    @functools.partial(
        pl.kernel,
        out_type=(part_sd, part_sd),
        mesh=mesh,
        scratch_types=[
            pltpu.VMEM((cpt, ch), jnp.int32),
            pltpu.VMEM((cpt, ch), jnp.int32),
            buf, buf, buf, buf, buf, buf, buf, buf,
            pltpu.VMEM_SHARED((n_pad, dh), jnp.float32),
            pltpu.SemaphoreType.DMA, pltpu.SemaphoreType.DMA,
            pltpu.SemaphoreType.DMA, pltpu.SemaphoreType.DMA,
            pltpu.SemaphoreType.DMA, pltpu.SemaphoreType.DMA,
            pltpu.SemaphoreType.DMA, pltpu.SemaphoreType.DMA,
        ],
        compiler_params=pltpu.CompilerParams(needs_layout_passes=False,
                                             use_tc_tiling_on_sc=False),
    )
    def run(hlo_hbm, hhi_hbm, ef_hbm, rowi_hbm, coli_hbm, z_hbm,
            outlo_hbm, outhi_hbm,
            row_v, col_v, r0, r1, r2, e0, e1, e2, m0, m1, aggr,
            gs0, gs1, gs2, es0, es1, es2, ss0, ss1):
        c = lax.axis_index("c")
        s = lax.axis_index("s")
        wid = c * NS + s
        mine = pl.ds(s * rps, rps)
        pltpu.sync_copy(rowi_hbm.at[wid], row_v)
        pltpu.sync_copy(coli_hbm.at[wid], col_v)
        fslots = ((r0, e0, gs0, es0), (r1, e1, gs1, es1), (r2, e2, gs2, es2))
        mslots = ((m0, ss0), (m1, ss1))

        for hx, (h_hbm, out_hbm) in enumerate(((hlo_hbm, outlo_hbm),
                                               (hhi_hbm, outhi_hbm))):
            off = pl.ds(hx * dh, dh)
            pltpu.sync_copy(z_hbm.at[mine], aggr.at[mine])
            plsc.subcore_barrier()

            def start_fetch(j, rows_v, ef_v, gsem, esem):
                pltpu.async_copy(h_hbm.at[row_v.at[j]], rows_v, gsem)
                pltpu.async_copy(
                    ef_hbm.at[pl.ds((wid * cpt + j) * ch, ch), off],
                    ef_v, esem)

            for b in range(NF):
                rows_v, ef_v, gsem, esem = fslots[b]
                start_fetch(b, rows_v, ef_v, gsem, esem)

            def six(j6, carry):
                for t in range(NF * NM):
                    rows_v, ef_v, gsem, esem = fslots[t % NF]
                    msg_v, ssem = mslots[t % NM]
                    j = NF * NM * j6 + t
                    pltpu.make_async_copy(h_hbm.at[row_v.at[j]], rows_v,
                                          gsem).wait()
                    pltpu.make_async_copy(
                        ef_hbm.at[pl.ds((wid * cpt + j) * ch, ch), off],
                        ef_v, esem).wait()

                    @pl.when(j >= NM)
                    def _():
                        pltpu.make_async_copy(msg_v, aggr.at[col_v.at[j]],
                                              ssem).wait()

                    def rloop(r4, c2):
                        for rr in range(4):
                            r = r4 * 4 + rr
                            for q in range(dh // LANES):
                                sl = pl.ds(q * LANES, LANES)
                                msg_v[r, sl] = rows_v[r, sl] * ef_v[r, sl]
                        return c2

                    lax.fori_loop(0, ch // 4, rloop, 0)

                    @pl.when(j + NF < cpt)
                    def _():
                        start_fetch(j + NF, rows_v, ef_v, gsem, esem)

                    pltpu.async_copy(msg_v, aggr.at[col_v.at[j]], ssem,
                                     add=True)
                return carry

            lax.fori_loop(0, cpt // (NF * NM), six, 0)
            for b in range(NM):
                msg_v, ssem = mslots[b]
                j = cpt - NM + b
                pltpu.make_async_copy(msg_v, aggr.at[col_v.at[j]], ssem).wait()
            plsc.subcore_barrier()
            pltpu.sync_copy(aggr.at[mine], out_hbm.at[c, mine])
            plsc.subcore_barrier()

    return run(h_lo, h_hi, ef, rowi, coli, zeros_h)


def _tc_ef(d2m, mu_col, We, be, e_pad, nrbf, d):
    eb = 8192
    sb = eb // 128
    grid = (e_pad // eb,)

    def body(d2_ref, mu_ref, w, b, o):
        cols = []
        for s in range(sb):
            dist = jnp.sqrt(d2_ref[s:s + 1, :] + 1e-12)
            delta = dist - mu_ref[...]
            cols.append(jnp.exp(-GAMMA * (delta * delta)))
        rbf_t = jnp.concatenate(cols, axis=1)
        y = lax.dot_general(rbf_t, w[...], (((0,), (0,)), ((), ())),
                            preferred_element_type=jnp.float32)
        yb = y + b[...]
        for s in range(sb):
            o[s] = yb[s * 128:(s + 1) * 128, :]

    full = lambda shape: pl.BlockSpec(shape, lambda i: (0,) * len(shape))
    return pl.pallas_call(
        body,
        grid=grid,
        in_specs=[
            pl.BlockSpec((sb, 128), lambda i: (i, 0)),
            full((nrbf, 1)),
            full((nrbf, d)),
            full((1, d)),
        ],
        out_specs=pl.BlockSpec((sb, 128, d), lambda i: (i, 0, 0)),
        out_shape=jax.ShapeDtypeStruct((e_pad // 128, 128, d), jnp.float32),
    )(d2m, mu_col, We, be)


def _tc_dense_in(a, W, b, n_pad, d, dh):
    rb = 512
    grid = (n_pad // rb,)

    def body(a_ref, w_ref, b_ref, ol_ref, oh_ref):
        y = (jnp.dot(a_ref[...], w_ref[...], preferred_element_type=jnp.float32)
             + b_ref[...])
        ol_ref[...] = y[:, :dh]
        oh_ref[...] = y[:, dh:]

    out_sd = jax.ShapeDtypeStruct((n_pad, dh), jnp.float32)
    return pl.pallas_call(
        body,
        grid=grid,
        in_specs=[
            pl.BlockSpec((rb, d), lambda i: (i, 0)),
            pl.BlockSpec((d, d), lambda i: (0, 0)),
            pl.BlockSpec((1, d), lambda i: (0, 0)),
        ],
        out_specs=[pl.BlockSpec((rb, dh), lambda i: (i, 0))] * 2,
        out_shape=[out_sd] * 2,
    )(a, W, b)


def _tc_dense_layer(p_lo, p_hi, W, b, n_pad, d, dh):
    rb = 512
    grid = (n_pad // rb,)

    def body(pl_ref, ph_ref, w_ref, b_ref, ol_ref, oh_ref):
        acc = jnp.concatenate(
            [jnp.sum(pl_ref[...], axis=0), jnp.sum(ph_ref[...], axis=0)],
            axis=1)
        y = jnp.dot(acc, w_ref[...], preferred_element_type=jnp.float32) + b_ref[...]
        y = jnp.maximum(y, 0.0)
        ol_ref[...] = y[:, :dh]
        oh_ref[...] = y[:, dh:]

    out_sd = jax.ShapeDtypeStruct((n_pad, dh), jnp.float32)
    return pl.pallas_call(
        body,
        grid=grid,
        in_specs=[
            pl.BlockSpec((NC, rb, dh), lambda i: (0, i, 0)),
            pl.BlockSpec((NC, rb, dh), lambda i: (0, i, 0)),
            pl.BlockSpec((d, d), lambda i: (0, 0)),
            pl.BlockSpec((1, d), lambda i: (0, 0)),
        ],
        out_specs=[pl.BlockSpec((rb, dh), lambda i: (i, 0))] * 2,
        out_shape=[out_sd] * 2,
    )(p_lo, p_hi, W, b)


def _tc_segmean(h_lo, h_hi, batch2, n_pad, d, dh):
    rb = 1024
    grid = (n_pad // rb,)

    def body(b_ref, hl_ref, hh_ref, o_ref, acc, cacc):
        i = pl.program_id(0)
        seg = lax.broadcasted_iota(jnp.int32, (NG, rb), 0)
        m = (b_ref[...] == seg).astype(jnp.float32)
        sl = jnp.dot(m, hl_ref[...], preferred_element_type=jnp.float32)
        sh = jnp.dot(m, hh_ref[...], preferred_element_type=jnp.float32)
        sums = jnp.concatenate([sl, sh], axis=1)
        cnt = jnp.sum(m, axis=1, keepdims=True)

        @pl.when(i == 0)
        def _():
            acc[...] = jnp.zeros_like(acc)
            cacc[...] = jnp.zeros_like(cacc)

        acc[...] = acc[...] + sums
        cacc[...] = cacc[...] + cnt

        @pl.when(i == grid[0] - 1)
        def _():
            o_ref[...] = acc[...] / jnp.maximum(cacc[...], 1.0)

    return pl.pallas_call(
        body,
        grid=grid,
        in_specs=[
            pl.BlockSpec((1, rb), lambda i: (0, i)),
            pl.BlockSpec((rb, dh), lambda i: (i, 0)),
            pl.BlockSpec((rb, dh), lambda i: (i, 0)),
        ],
        out_specs=pl.BlockSpec((NG, d), lambda i: (0, 0)),
        out_shape=jax.ShapeDtypeStruct((NG, d), jnp.float32),
        scratch_shapes=[
            pltpu.VMEM((NG, d), jnp.float32),
            pltpu.VMEM((NG, 1), jnp.float32),
        ],
    )(batch2, h_lo, h_hi)


def kernel(x, pos, edge_index, edge_attr, batch, W_in, b_in, mu,
           We1, be1, W1, b1, We2, be2, W2, b2, We3, be3, W3, b3):
    n, d = x.shape
    e = edge_index.shape[1]
    nrbf = mu.shape[0]
    dh = d // 2

    ch = 128
    cpt = -(-e // (NW * ch))
    cpt = -(-cpt // 6) * 6
    e_pad = NW * cpt * ch
    n_pad = -(-(n + 1) // 2048) * 2048

    row = edge_index[0]
    col = edge_index[1]
    e_pt = -(-e // NW)
    row_p = jnp.concatenate([row, jnp.zeros((e_pt * NW - e,), jnp.int32)])
    col_p = jnp.concatenate([col, jnp.full((e_pt * NW - e,), n, jnp.int32)])
    ep = cpt * ch
    pad_rows = jnp.broadcast_to(
        jnp.arange(ep - e_pt, dtype=jnp.int32) % n, (NW, ep - e_pt))
    pad_cols = jnp.broadcast_to(
        n + jnp.arange(ep - e_pt, dtype=jnp.int32) % (n_pad - n),
        (NW, ep - e_pt))
    rowi = jnp.concatenate(
        [row_p.reshape(NW, e_pt), pad_rows], axis=1).reshape(NW, cpt, ch)
    coli = jnp.concatenate(
        [col_p.reshape(NW, e_pt), pad_cols], axis=1).reshape(NW, cpt, ch)
    posx = jnp.pad(pos[:, 0], (0, n_pad - n))
    posy = jnp.pad(pos[:, 1], (0, n_pad - n))
    posz = jnp.pad(pos[:, 2], (0, n_pad - n))
    x_pad = jnp.pad(x, ((0, n_pad - n), (0, 0)))
    zeros_h = jnp.zeros((n_pad, dh), jnp.float32)
    batch2 = jnp.pad(batch, (0, n_pad - n), constant_values=NG).reshape(1, n_pad)
    mu_col = mu.reshape(nrbf, 1)

    d2 = _sc_d2(posx, posy, posz, rowi, coli, n_pad, cpt, ch)
    d2m = d2.reshape(e_pad // 128, 128)

    h_lo, h_hi = _tc_dense_in(x_pad, W_in, b_in.reshape(1, d), n_pad, d, dh)
    for We, be, W, b in ((We1, be1, W1, b1), (We2, be2, W2, b2),
                         (We3, be3, W3, b3)):
        ef = _tc_ef(d2m, mu_col, We, be.reshape(1, d), e_pad, nrbf, d)
        p_lo, p_hi = _sc_conv(h_lo, h_hi, ef.reshape(e_pad, d), rowi, coli,
                              zeros_h, n_pad, cpt, ch, d, dh)
        h_lo, h_hi = _tc_dense_layer(p_lo, p_hi, W, b.reshape(1, d),
                                     n_pad, d, dh)

    return _tc_segmean(h_lo, h_hi, batch2, n_pad, d, dh)
